# baseline (speedup 1.0000x reference)
import functools

import jax
import jax.numpy as jnp
from jax import lax
from jax.experimental import pallas as pl
from jax.experimental.pallas import tpu as pltpu
from jax.experimental.pallas import tpu_sc as plsc

_EPS = 1e-5
_NC, _NS = 2, 16
_NW = _NC * _NS
_K = 128


def _sc_gather_make(N, D, E):
    NCH = E // _K
    NCH2 = N // _K
    REM2 = N - NCH2 * _K
    mesh = plsc.VectorSubcoreMesh(core_axis_name="c", subcore_axis_name="s")
    sd = jax.ShapeDtypeStruct

    @functools.partial(
        pl.kernel,
        out_type=[sd((E, D), jnp.float32),
                  sd((N, D), jnp.float32),
                  sd((_NW, N), jnp.int32),
                  sd((_NC, N), jnp.float32)],
        mesh=mesh,
    scratch_types=[
            pltpu.VMEM((_K,), jnp.int32),
            pltpu.VMEM((_K,), jnp.int32),
            pltpu.VMEM((_K,), jnp.int32),
            pltpu.VMEM((_K,), jnp.int32),
            pltpu.VMEM((_K, D), jnp.float32),
            pltpu.VMEM((_K, D), jnp.float32),
            pltpu.VMEM((_K, D), jnp.float32),
            pltpu.VMEM((_K, D), jnp.float32),
            pltpu.VMEM((_K,), jnp.int32),
            pltpu.VMEM((_K,), jnp.int32),
            pltpu.VMEM((_K,), jnp.int32),
            pltpu.VMEM((_K,), jnp.int32),
            pltpu.VMEM((_K,), jnp.int32),
            pltpu.VMEM((_K,), jnp.float32),
            pltpu.VMEM((N,), jnp.int32),
            pltpu.VMEM_SHARED((_NS * N,), jnp.int32),
            pltpu.VMEM_SHARED((N,), jnp.float32),
            pltpu.SemaphoreType.DMA,
            pltpu.SemaphoreType.DMA,
            pltpu.SemaphoreType.DMA,
            pltpu.SemaphoreType.DMA,
            pltpu.SemaphoreType.DMA,
            pltpu.SemaphoreType.DMA,
            pltpu.SemaphoreType.DMA,
            pltpu.SemaphoreType.DMA,
            pltpu.SemaphoreType.DMA,
            pltpu.SemaphoreType.DMA,
        ],
    )
    def k(linh_hbm, hpws_hbm, send_hbm, rec_hbm, zeros_hbm,
          g_hbm, g2_hbm, win_hbm, cnt_hbm,
          sidx0, sidx1, ridx0, ridx1, rowsa0, rowsa1, rowsb0, rowsb1,
          eid0, eid1, sidx20, sidx21, neg_v, ones_v,
          win_v, win_sh, cnt_sh, sema0, sema1, semb0, semb1, semo0, semo1,
          semw0, semw1, semc0, semc1):
        sidx_s = (sidx0, sidx1)
        ridx_s = (ridx0, ridx1)
        rowsa_s = (rowsa0, rowsa1)
        rowsb_s = (rowsb0, rowsb1)
        eid_s = (eid0, eid1)
        sidx2_s = (sidx20, sidx21)
        sema_s = (sema0, sema1)
        semb_s = (semb0, semb1)
        semo_s = (semo0, semo1)
        semw_s = (semw0, semw1)
        semc_s = (semc0, semc1)
        cid = lax.axis_index("c")
        sid = lax.axis_index("s")
        wid = sid * _NC + cid

        @pl.when(sid == 0)
        def _init_cnt():
            pltpu.sync_copy(zeros_hbm, cnt_sh)

        for i in range(_K // 16):
            sl = pl.ds(i * 16, 16)
            ones_v[sl] = jnp.ones((16,), jnp.float32)
            neg_v[sl] = jnp.full((16,), -1, jnp.int32)

        def _fill_win(j, c):
            pltpu.sync_copy(neg_v, win_sh.at[pl.ds(sid * N + j * _K, _K)])
            return c
        lax.fori_loop(0, N // _K, _fill_win, 0)
        if N % _K > 0:
            pltpu.sync_copy(neg_v.at[pl.ds(0, N % _K)],
                            win_sh.at[pl.ds(sid * N + (N // _K) * _K, N % _K)])
        plsc.subcore_barrier()

        n_i = (NCH - wid + _NW - 1) // _NW
        cbase = lambda t: (wid + t * _NW) * _K

        @pl.when(n_i > 0)
        def _prologue():
            pltpu.sync_copy(send_hbm.at[pl.ds(cbase(0), _K)], sidx0)
            pltpu.sync_copy(rec_hbm.at[pl.ds(cbase(0), _K)], ridx0)
            pltpu.async_copy(linh_hbm.at[sidx0], rowsa0, sema0)
            pltpu.async_copy(linh_hbm.at[ridx0], rowsb0, semb0)

        def _pair(j, c):
            for b in (0, 1):
                t = 2 * j + b
                bn = 1 - b

                @pl.when(t < n_i)
                def _step():
                    base = cbase(t)

                    @pl.when(t + 1 < n_i)
                    def _prefetch():
                        nbase = cbase(t + 1)

                        @pl.when(t + 1 >= 2)
                        def _drain_prev():
                            pltpu.make_async_copy(
                                rowsa_s[bn], g_hbm.at[pl.ds(base, _K)],
                                semo_s[bn]).wait()
                            pltpu.make_async_copy(
                                ones_v, cnt_sh.at[sidx_s[bn]],
                                semc_s[bn]).wait()
                        pltpu.sync_copy(send_hbm.at[pl.ds(nbase, _K)],
                                        sidx_s[bn])
                        pltpu.sync_copy(rec_hbm.at[pl.ds(nbase, _K)],
                                        ridx_s[bn])
                        pltpu.async_copy(linh_hbm.at[sidx_s[bn]],
                                         rowsa_s[bn], sema_s[bn])
                        pltpu.async_copy(linh_hbm.at[ridx_s[bn]],
                                         rowsb_s[bn], semb_s[bn])

                    pltpu.make_async_copy(linh_hbm.at[sidx_s[b]],
                                          rowsa_s[b], sema_s[b]).wait()
                    pltpu.make_async_copy(linh_hbm.at[ridx_s[b]],
                                          rowsb_s[b], semb_s[b]).wait()
                    ra = rowsa_s[b]
                    rb = rowsb_s[b]

                    def _radd(r, cc):
                        for rr in range(2):
                            for c8 in range(D // 16):
                                sl = pl.ds(c8 * 16, 16)
                                ra[2 * r + rr, sl] = (ra[2 * r + rr, sl]
                                                      + rb[2 * r + rr, sl])
                        return cc
                    lax.fori_loop(0, _K // 2, _radd, 0)
                    pltpu.async_copy(ra, g_hbm.at[pl.ds(base, _K)],
                                     semo_s[b])
                    for r8 in range(_K // 16):
                        sl = pl.ds(r8 * 16, 16)
                        eid_s[b][sl] = base + r8 * 16 + lax.iota(jnp.int32, 16)
                        sidx2_s[b][sl] = sidx_s[b][sl] + sid * N
                    pltpu.sync_copy(eid_s[b], win_sh.at[sidx2_s[b]])
                    pltpu.async_copy(ones_v, cnt_sh.at[sidx_s[b]],
                                     semc_s[b], add=True)
            return c

        lax.fori_loop(0, (n_i + 1) // 2, _pair, 0)
        for b in (0, 1):
            lb = jnp.where((n_i - 1) % 2 == b, n_i - 1, n_i - 2)

            @pl.when(lb >= 0)
            def _drain_tail():
                pltpu.make_async_copy(rowsa_s[b],
                                      g_hbm.at[pl.ds(cbase(0), _K)],
                                      semo_s[b]).wait()
                pltpu.make_async_copy(ones_v, cnt_sh.at[sidx_s[b]],
                                      semc_s[b]).wait()

        def _chunk2(i, c):
            base = (wid + i * _NW) * _K
            pltpu.sync_copy(send_hbm.at[pl.ds(base, _K)], sidx0)
            pltpu.async_copy(hpws_hbm.at[sidx0], rowsa0, sema0).wait()
            pltpu.sync_copy(rowsa0, g2_hbm.at[pl.ds(base, _K)])
            return c

        n_i2 = (NCH2 - wid + _NW - 1) // _NW
        lax.fori_loop(0, n_i2, _chunk2, 0)

        if REM2 > 0:
            @pl.when(wid == _NW - 1)
            def _tail():
                tb = NCH2 * _K
                pltpu.sync_copy(send_hbm.at[pl.ds(tb, REM2)],
                                sidx0.at[pl.ds(0, REM2)])
                pltpu.async_copy(hpws_hbm.at[sidx0.at[pl.ds(0, REM2)]],
                                 rowsa0.at[pl.ds(0, REM2)], sema0).wait()
                pltpu.sync_copy(rowsa0.at[pl.ds(0, REM2)],
                                g2_hbm.at[pl.ds(tb, REM2)])

        pltpu.sync_copy(win_sh.at[pl.ds(sid * N, N)], win_v)
        pltpu.sync_copy(win_v, win_hbm.at[wid])
        plsc.subcore_barrier()

        @pl.when(sid == 0)
        def _write_cnt():
            pltpu.sync_copy(cnt_sh, cnt_hbm.at[cid])

    return k


def _sc_segsum_make(N, D, E):
    NCH = E // _K
    ZR = 125
    mesh = plsc.VectorSubcoreMesh(core_axis_name="c", subcore_axis_name="s")

    @functools.partial(
        pl.kernel,
        out_type=jax.ShapeDtypeStruct((_NC, N, D), jnp.float32),
        mesh=mesh,
        scratch_types=[
            pltpu.VMEM((_K,), jnp.int32),
            pltpu.VMEM((_K,), jnp.int32),
            pltpu.VMEM((_K, D), jnp.float32),
            pltpu.VMEM((_K, D), jnp.float32),
            pltpu.VMEM((ZR, D), jnp.float32),
            pltpu.VMEM_SHARED((N, D), jnp.float32),
            pltpu.SemaphoreType.DMA,
            pltpu.SemaphoreType.DMA,
            pltpu.SemaphoreType.DMA,
            pltpu.SemaphoreType.DMA,
        ],
    )
    def k(etanew_hbm, rec_hbm, w_hbm, ridx0, ridx1, env0, env1, zv, w_sh,
          seme0, seme1, sems0, sems1):
        ridx_s = (ridx0, ridx1)
        env_s = (env0, env1)
        seme_s = (seme0, seme1)
        sems_s = (sems0, sems1)
        cid = lax.axis_index("c")
        sid = lax.axis_index("s")
        wid = sid * _NC + cid

        def _zrow(r, c):
            for c8 in range(D // 16):
                zv[r, pl.ds(c8 * 16, 16)] = jnp.zeros((16,), jnp.float32)
            return c
        lax.fori_loop(0, ZR, _zrow, 0)
        nstripe = N // (_NS * ZR)
        def _zcp(j, c):
            pltpu.sync_copy(zv, w_sh.at[pl.ds((sid * nstripe + j) * ZR, ZR)])
            return c
        lax.fori_loop(0, nstripe, _zcp, 0)
        plsc.subcore_barrier()

        n_i = (NCH - wid + _NW - 1) // _NW
        cbase = lambda t: (wid + t * _NW) * _K

        @pl.when(n_i > 0)
        def _prologue():
            pltpu.sync_copy(rec_hbm.at[pl.ds(cbase(0), _K)], ridx0)
            pltpu.async_copy(etanew_hbm.at[pl.ds(cbase(0), _K)], env0, seme0)

        def _pair(j, c):
            for b in (0, 1):
                t = 2 * j + b
                bn = 1 - b

                @pl.when(t < n_i)
                def _step():
                    @pl.when(t + 1 < n_i)
                    def _prefetch():
                        nbase = cbase(t + 1)

                        @pl.when(t + 1 >= 2)
                        def _drain_prev():
                            pltpu.make_async_copy(
                                env_s[bn], w_sh.at[ridx_s[bn]],
                                sems_s[bn]).wait()
                        pltpu.sync_copy(rec_hbm.at[pl.ds(nbase, _K)],
                                        ridx_s[bn])
                        pltpu.async_copy(etanew_hbm.at[pl.ds(nbase, _K)],
                                         env_s[bn], seme_s[bn])

                    pltpu.make_async_copy(etanew_hbm.at[pl.ds(cbase(t), _K)],
                                          env_s[b], seme_s[b]).wait()
                    pltpu.async_copy(env_s[b], w_sh.at[ridx_s[b]],
                                     sems_s[b], add=True)
            return c

        lax.fori_loop(0, (n_i + 1) // 2, _pair, 0)
        for b in (0, 1):
            lb = jnp.where((n_i - 1) % 2 == b, n_i - 1, n_i - 2)

            @pl.when(lb >= 0)
            def _drain_tail():
                pltpu.make_async_copy(env_s[b], w_sh.at[ridx_s[b]],
                                      sems_s[b]).wait()
        plsc.subcore_barrier()

        @pl.when(sid == 0)
        def _write():
            pltpu.sync_copy(w_sh, w_hbm.at[cid])

    return k


def _sc_gather2_make(N, D):
    NCH2 = N // _K
    REM2 = N - NCH2 * _K
    mesh = plsc.VectorSubcoreMesh(core_axis_name="c", subcore_axis_name="s")
    sd = jax.ShapeDtypeStruct

    @functools.partial(
        pl.kernel,
        out_type=[sd((N, D), jnp.float32), sd((N, D), jnp.float32)],
        mesh=mesh,
        scratch_types=[
            pltpu.VMEM((_K,), jnp.int32),
            pltpu.VMEM((_K,), jnp.int32),
            pltpu.VMEM((_K, D), jnp.float32),
            pltpu.VMEM((_K, D), jnp.float32),
            pltpu.VMEM((_K, D), jnp.float32),
            pltpu.VMEM((_K, D), jnp.float32),
            pltpu.SemaphoreType.DMA,
            pltpu.SemaphoreType.DMA,
            pltpu.SemaphoreType.DMA,
            pltpu.SemaphoreType.DMA,
        ],
    )
    def k(seg1_hbm, seg2_hbm, wc_hbm, s1g_hbm, s2g_hbm,
          idx0, idx1, r10, r11, r20, r21, semg0, semg1, semo0, semo1):
        idx_s = (idx0, idx1)
        r1_s = (r10, r11)
        r2_s = (r20, r21)
        semg_s = (semg0, semg1)
        semo_s = (semo0, semo1)
        cid = lax.axis_index("c")
        sid = lax.axis_index("s")
        wid = sid * _NC + cid

        n_i = (NCH2 - wid + _NW - 1) // _NW
        cbase = lambda t: (wid + t * _NW) * _K

        @pl.when(n_i > 0)
        def _prologue():
            pltpu.sync_copy(wc_hbm.at[pl.ds(cbase(0), _K)], idx0)
            pltpu.async_copy(seg1_hbm.at[idx0], r10, semg0)
            pltpu.async_copy(seg2_hbm.at[idx0], r20, semg0)

        def _pair(j, c):
            for b in (0, 1):
                t = 2 * j + b
                bn = 1 - b

                @pl.when(t < n_i)
                def _step():
                    base = cbase(t)

                    @pl.when(t + 1 < n_i)
                    def _prefetch():
                        nbase = cbase(t + 1)

                        @pl.when(t + 1 >= 2)
                        def _drain_prev():
                            pltpu.make_async_copy(
                                r1_s[bn], s1g_hbm.at[pl.ds(base, _K)],
                                semo_s[bn]).wait()
                            pltpu.make_async_copy(
                                r2_s[bn], s2g_hbm.at[pl.ds(base, _K)],
                                semo_s[bn]).wait()
                        pltpu.sync_copy(wc_hbm.at[pl.ds(nbase, _K)],
                                        idx_s[bn])
                        pltpu.async_copy(seg1_hbm.at[idx_s[bn]],
                                         r1_s[bn], semg_s[bn])
                        pltpu.async_copy(seg2_hbm.at[idx_s[bn]],
                                         r2_s[bn], semg_s[bn])

                    pltpu.make_async_copy(seg1_hbm.at[idx_s[b]],
                                          r1_s[b], semg_s[b]).wait()
                    pltpu.make_async_copy(seg2_hbm.at[idx_s[b]],
                                          r2_s[b], semg_s[b]).wait()
                    pltpu.async_copy(r1_s[b], s1g_hbm.at[pl.ds(base, _K)],
                                     semo_s[b])
                    pltpu.async_copy(r2_s[b], s2g_hbm.at[pl.ds(base, _K)],
                                     semo_s[b])
            return c

        lax.fori_loop(0, (n_i + 1) // 2, _pair, 0)
        for b in (0, 1):
            lb = jnp.where((n_i - 1) % 2 == b, n_i - 1, n_i - 2)

            @pl.when(lb >= 0)
            def _drain_tail():
                pltpu.make_async_copy(r1_s[b], s1g_hbm.at[pl.ds(cbase(0), _K)],
                                      semo_s[b]).wait()
                pltpu.make_async_copy(r2_s[b], s2g_hbm.at[pl.ds(cbase(0), _K)],
                                      semo_s[b]).wait()

        if REM2 > 0:
            @pl.when(wid == _NW - 1)
            def _tail():
                tb = NCH2 * _K
                pltpu.sync_copy(wc_hbm.at[pl.ds(tb, REM2)],
                                idx0.at[pl.ds(0, REM2)])
                cp_a = pltpu.async_copy(seg1_hbm.at[idx0.at[pl.ds(0, REM2)]],
                                        r10.at[pl.ds(0, REM2)], semg0)
                cp_b = pltpu.async_copy(seg2_hbm.at[idx0.at[pl.ds(0, REM2)]],
                                        r20.at[pl.ds(0, REM2)], semg0)
                cp_a.wait()
                cp_b.wait()
                pltpu.sync_copy(r10.at[pl.ds(0, REM2)],
                                s1g_hbm.at[pl.ds(tb, REM2)])
                pltpu.sync_copy(r20.at[pl.ds(0, REM2)],
                                s2g_hbm.at[pl.ds(tb, REM2)])

    return k


def _combine_body(winp_ref, cntp_ref, win_ref, cnt_ref):
    win_ref[...] = jnp.max(winp_ref[...], axis=0, keepdims=True)
    cnt_ref[...] = jnp.sum(cntp_ref[...], axis=0, keepdims=True)


def _combine(win_parts, cnt_parts):
    N = win_parts.shape[1]
    full = lambda shape: pl.BlockSpec(shape, lambda: (0, 0))
    return pl.pallas_call(
        _combine_body,
        in_specs=[full((_NW, N)), full((_NC, N))],
        out_specs=[full((1, N)), full((1, N))],
        out_shape=[jax.ShapeDtypeStruct((1, N), jnp.int32),
                   jax.ShapeDtypeStruct((1, N), jnp.float32)],
    )(win_parts, cnt_parts)


def _node_mm_body(h_ref, p_ref, Wlin_ref, blin_ref, Ws_ref, bs_ref, Wr_ref,
                  br_ref, Wp1_ref, bp1_ref, Wp2_ref, bp2_ref,
                  linh_ref, hpWs_ref, hpWr_ref, pWp1_ref, pWp2_ref):
    hb = h_ref[...]
    pb = p_ref[...]
    hp = jnp.concatenate([hb, pb], axis=1)
    f32 = jnp.float32
    linh_ref[...] = jnp.dot(hb, Wlin_ref[...], preferred_element_type=f32) + blin_ref[...]
    hpWs_ref[...] = jnp.dot(hp, Ws_ref[...], preferred_element_type=f32) + bs_ref[...]
    hpWr_ref[...] = jnp.dot(hp, Wr_ref[...], preferred_element_type=f32) + br_ref[...]
    pWp1_ref[...] = jnp.dot(pb, Wp1_ref[...], preferred_element_type=f32) + bp1_ref[...]
    pWp2_ref[...] = jnp.dot(pb, Wp2_ref[...], preferred_element_type=f32) + bp2_ref[...]


def _node_matmuls(h, p, W_lin, b_lin, Ws, bs, Wr, br, Wp1, bp1, Wp2, bp2):
    N, D = h.shape
    R = 1000
    grid = (N // R,)
    row = pl.BlockSpec((R, D), lambda i: (i, 0))
    row2 = pl.BlockSpec((R, 2 * D), lambda i: (i, 0))
    wfull = lambda shape: pl.BlockSpec(shape, lambda i: (0, 0))
    out_sd = jax.ShapeDtypeStruct((N, D), jnp.float32)
    return pl.pallas_call(
        _node_mm_body,
        grid=grid,
        in_specs=[row, row,
                  wfull((D, D)), wfull((1, D)),
                  wfull((2 * D, D)), wfull((1, D)),
                  wfull((2 * D, D)), wfull((1, D)),
                  wfull((D, D)), wfull((1, D)),
                  wfull((D, D)), wfull((1, D))],
        out_specs=[row, row, row, row, row],
        out_shape=[out_sd] * 5,
    )(h, p, W_lin, b_lin.reshape(1, D), Ws, bs.reshape(1, D),
      Wr, br.reshape(1, D), Wp1, bp1.reshape(1, D), Wp2, bp2.reshape(1, D))


def _edge_eta_body(g_ref, e_ref, Wlin_ref, b3_ref,
                   etanew_ref, s_ref, sum_ref, sumsq_ref):
    x = g_ref[...] + jnp.dot(e_ref[...], Wlin_ref[...],
                             preferred_element_type=jnp.float32) + b3_ref[...]
    eta = jax.nn.sigmoid(x)
    s = jnp.sum(eta, axis=1, keepdims=True)
    etanew_ref[...] = eta / s
    s_ref[...] = s
    bsum = jnp.sum(eta, axis=0, keepdims=True)
    bsq = jnp.sum(eta * eta, axis=0, keepdims=True)

    @pl.when(pl.program_id(0) == 0)
    def _init():
        sum_ref[...] = bsum
        sumsq_ref[...] = bsq

    @pl.when(pl.program_id(0) != 0)
    def _acc():
        sum_ref[...] += bsum
        sumsq_ref[...] += bsq


def _edge_eta(g, e, W_lin, b_lin):
    E, D = e.shape
    BE = 1000
    grid = (E // BE,)
    row = pl.BlockSpec((BE, D), lambda i: (i, 0))
    col = pl.BlockSpec((BE, 1), lambda i: (i, 0))
    wfull = lambda shape: pl.BlockSpec(shape, lambda i: (0, 0))
    return pl.pallas_call(
        _edge_eta_body,
        grid=grid,
        in_specs=[row, row, wfull((D, D)), wfull((1, D))],
        out_specs=[row, col, wfull((1, D)), wfull((1, D))],
        out_shape=[jax.ShapeDtypeStruct((E, D), jnp.float32),
                   jax.ShapeDtypeStruct((E, 1), jnp.float32),
                   jax.ShapeDtypeStruct((1, D), jnp.float32),
                   jax.ShapeDtypeStruct((1, D), jnp.float32)],
    )(g, e, W_lin, b_lin.reshape(1, D))


def _stats_body(W0_ref, W1_ref, cnt_ref, win_ref, g2_ref, hpWs_ref,
                hpWr_ref, pWp2_ref, sume_ref, sumsqe_ref, n_edges_ref,
                seg1_ref, seg2_ref, mean1_ref, inv1_ref,
                meane_ref, inve_ref, wc_ref):
    E = n_edges_ref[0]
    Nn = n_edges_ref[1]
    Ef = E.astype(jnp.float32)
    W = W0_ref[...] + W1_ref[...]
    wc_ref[...] = jnp.clip(win_ref[...], 0, Nn - 1)
    seg1 = hpWr_ref[...] * W
    seg2 = pWp2_ref[...] * W
    seg1_ref[...] = seg1
    seg2_ref[...] = seg2
    cnt = cnt_ref[...]
    hpWs = hpWs_ref[...]
    g2 = g2_ref[...]
    A1 = jnp.sum(cnt * hpWs, axis=0, keepdims=True)
    B1 = jnp.sum(cnt * hpWs * hpWs, axis=0, keepdims=True)
    S1 = jnp.sum(seg1, axis=0, keepdims=True)
    C1 = jnp.sum(2.0 * g2 * seg1 + seg1 * seg1, axis=0, keepdims=True)
    part1 = A1 + S1
    part2 = B1 + C1

    @pl.when(pl.program_id(0) == 0)
    def _init():
        mean1_ref[...] = part1
        inv1_ref[...] = part2

    @pl.when(pl.program_id(0) != 0)
    def _acc():
        mean1_ref[...] += part1
        inv1_ref[...] += part2

    @pl.when(pl.program_id(0) == pl.num_programs(0) - 1)
    def _fin():
        mean1 = mean1_ref[...] / Ef
        var1 = inv1_ref[...] / Ef - mean1 * mean1
        mean1_ref[...] = mean1
        inv1_ref[...] = lax.rsqrt(var1 + _EPS)
        meane = sume_ref[...] / Ef
        vare = sumsqe_ref[...] / Ef - meane * meane
        meane_ref[...] = meane
        inve_ref[...] = lax.rsqrt(vare + _EPS)


def _stats(W0, W1, cnt, win, g2, hpWs, hpWr, pWp2, sum_eta, sumsq_eta, E):
    N, D = W0.shape
    R = 2000
    grid = (N // R,)
    row = pl.BlockSpec((R, D), lambda i: (i, 0))
    col = pl.BlockSpec((R, 1), lambda i: (i, 0))
    bc = pl.BlockSpec((1, D), lambda i: (0, 0))
    sd = jax.ShapeDtypeStruct
    return pl.pallas_call(
        _stats_body,
        grid=grid,
        in_specs=[row, row, col, col, row, row, row, row, bc, bc,
                  pl.BlockSpec(memory_space=pltpu.SMEM)],
        out_specs=[row, row, bc, bc, bc, bc, col],
        out_shape=[sd((N, D), jnp.float32), sd((N, D), jnp.float32),
                   sd((1, D), jnp.float32), sd((1, D), jnp.float32),
                   sd((1, D), jnp.float32), sd((1, D), jnp.float32),
                   sd((N, 1), jnp.int32)],
    )(W0, W1, cnt.reshape(N, 1), win.reshape(N, 1), g2, hpWs, hpWr, pWp2,
      sum_eta, sumsq_eta, jnp.array([E, N], dtype=jnp.int32))


def _node_out_body(h_ref, p_ref, hpWs_ref, pWp1_ref, s1g_ref, s2g_ref,
                   win_ref, mean1_ref, inv1_ref, gamma_ref, beta_ref,
                   n_ref, hout_ref, pout_ref):
    Nn = n_ref[0]
    win = win_ref[...]
    has = win >= 0
    use = jnp.logical_and(has, win < Nn)
    h = h_ref[...]
    p = p_ref[...]
    x1 = hpWs_ref[...] + jnp.where(use, s1g_ref[...], 0.0)
    bn1 = (x1 - mean1_ref[...]) * inv1_ref[...] * gamma_ref[...] + beta_ref[...]
    hn = h + jnp.maximum(bn1, 0.0)
    hout_ref[...] = jnp.where(has, hn, h)
    x2 = pWp1_ref[...] + jnp.where(use, s2g_ref[...], 0.0)
    pout_ref[...] = jnp.where(has, p + jnp.tanh(x2), p)


def _node_out(h, p, hpWs, pWp1, s1g, s2g, win, mean1, inv1, gamma, beta, N_dim):
    N, D = h.shape
    R = 1000
    grid = (N // R,)
    row = pl.BlockSpec((R, D), lambda i: (i, 0))
    col = pl.BlockSpec((R, 1), lambda i: (i, 0))
    bc = lambda shape: pl.BlockSpec(shape, lambda i: (0, 0))
    sd = jax.ShapeDtypeStruct
    return pl.pallas_call(
        _node_out_body,
        grid=grid,
        in_specs=[row, row, row, row, row, row, col,
                  bc((1, D)), bc((1, D)), bc((1, D)), bc((1, D)),
                  pl.BlockSpec(memory_space=pltpu.SMEM)],
        out_specs=[row, row],
        out_shape=[sd((N, D), jnp.float32), sd((N, D), jnp.float32)],
    )(h, p, hpWs, pWp1, s1g, s2g, win.reshape(N, 1), mean1, inv1,
      gamma.reshape(1, D), beta.reshape(1, D),
      jnp.array([N_dim], dtype=jnp.int32))


def _e_out_body(e_ref, en_ref, s_ref, meane_ref, inve_ref, gamma_ref,
                beta_ref, eout_ref):
    eta = en_ref[...] * s_ref[...]
    bn = (eta - meane_ref[...]) * inve_ref[...] * gamma_ref[...] + beta_ref[...]
    eout_ref[...] = e_ref[...] + jnp.maximum(bn, 0.0)


def _e_out(e, eta_new, s, meane, inve, gamma, beta):
    E, D = e.shape
    BE = 1000
    grid = (E // BE,)
    row = pl.BlockSpec((BE, D), lambda i: (i, 0))
    col = pl.BlockSpec((BE, 1), lambda i: (i, 0))
    bc = lambda shape: pl.BlockSpec(shape, lambda i: (0, 0))
    return pl.pallas_call(
        _e_out_body,
        grid=grid,
        in_specs=[row, row, col, bc((1, D)), bc((1, D)), bc((1, D)), bc((1, D))],
        out_specs=row,
        out_shape=jax.ShapeDtypeStruct((E, D), jnp.float32),
    )(e, eta_new, s, meane, inve, gamma.reshape(1, D), beta.reshape(1, D))


def kernel(h, e, p, edge_index, W_lin, b_lin, Ws, bs, Wr, br, Wp1, bp1,
           Wp2, bp2, gamma, beta):
    N, D = h.shape
    E = e.shape[0]
    send = edge_index[0]
    rec = edge_index[1]

    lin_h, hpWs, hpWr, pWp1, pWp2 = _node_matmuls(
        h, p, W_lin, b_lin, Ws, bs, Wr, br, Wp1, bp1, Wp2, bp2)

    zeros_n = jnp.zeros((N,), jnp.float32)
    g, g2, win_parts, cnt_parts = _sc_gather_make(N, D, E)(
        lin_h, hpWs, send, rec, zeros_n)
    win2, cnt2 = _combine(win_parts, cnt_parts)
    win = win2.reshape(N)
    cnt = cnt2.reshape(N)

    eta_new, s, sum_eta, sumsq_eta = _edge_eta(g, e, W_lin, b_lin)

    W_parts = _sc_segsum_make(N, D, E)(eta_new, rec)

    seg1, seg2, mean1, inv1, meane, inve, wc = _stats(
        W_parts[0], W_parts[1], cnt, win, g2, hpWs, hpWr, pWp2,
        sum_eta, sumsq_eta, E)

    s1g, s2g = _sc_gather2_make(N, D)(seg1, seg2, wc.reshape(N))

    h_out, p_out = _node_out(h, p, hpWs, pWp1, s1g, s2g, win, mean1, inv1,
                             gamma, beta, N)
    e_out = _e_out(e, eta_new, s, meane, inve, gamma, beta)
    return (h_out, e_out, p_out)


# speedup vs baseline: 1.0018x; 1.0018x over previous
import functools

import jax
import jax.numpy as jnp
from jax import lax
from jax.experimental import pallas as pl
from jax.experimental.pallas import tpu as pltpu
from jax.experimental.pallas import tpu_sc as plsc

_EPS = 1e-5
_NC, _NS = 2, 16
_NW = _NC * _NS
_K = 128


def _sc_gather_make(N, D, E):
    NCH = E // _K
    NCH2 = N // _K
    REM2 = N - NCH2 * _K
    mesh = plsc.VectorSubcoreMesh(core_axis_name="c", subcore_axis_name="s")
    sd = jax.ShapeDtypeStruct

    @functools.partial(
        pl.kernel,
        out_type=[sd((E, D), jnp.float32),
                  sd((N, D), jnp.float32),
                  sd((_NW, N), jnp.int32),
                  sd((_NC, N), jnp.float32)],
        mesh=mesh,
    scratch_types=[
            pltpu.VMEM((_K,), jnp.int32),
            pltpu.VMEM((_K,), jnp.int32),
            pltpu.VMEM((_K,), jnp.int32),
            pltpu.VMEM((_K,), jnp.int32),
            pltpu.VMEM((_K, D), jnp.float32),
            pltpu.VMEM((_K, D), jnp.float32),
            pltpu.VMEM((_K, D), jnp.float32),
            pltpu.VMEM((_K, D), jnp.float32),
            pltpu.VMEM((_K,), jnp.int32),
            pltpu.VMEM((_K,), jnp.int32),
            pltpu.VMEM((_K,), jnp.int32),
            pltpu.VMEM((_K,), jnp.int32),
            pltpu.VMEM((_K,), jnp.int32),
            pltpu.VMEM((_K,), jnp.float32),
            pltpu.VMEM((N,), jnp.int32),
            pltpu.VMEM_SHARED((_NS * N,), jnp.int32),
            pltpu.VMEM_SHARED((N,), jnp.float32),
            pltpu.SemaphoreType.DMA,
            pltpu.SemaphoreType.DMA,
            pltpu.SemaphoreType.DMA,
            pltpu.SemaphoreType.DMA,
            pltpu.SemaphoreType.DMA,
            pltpu.SemaphoreType.DMA,
            pltpu.SemaphoreType.DMA,
            pltpu.SemaphoreType.DMA,
            pltpu.SemaphoreType.DMA,
            pltpu.SemaphoreType.DMA,
        ],
    )
    def k(linh_hbm, hpws_hbm, send_hbm, rec_hbm, zeros_hbm,
          g_hbm, g2_hbm, win_hbm, cnt_hbm,
          sidx0, sidx1, ridx0, ridx1, rowsa0, rowsa1, rowsb0, rowsb1,
          eid0, eid1, sidx20, sidx21, neg_v, ones_v,
          win_v, win_sh, cnt_sh, sema0, sema1, semb0, semb1, semo0, semo1,
          semw0, semw1, semc0, semc1):
        sidx_s = (sidx0, sidx1)
        ridx_s = (ridx0, ridx1)
        rowsa_s = (rowsa0, rowsa1)
        rowsb_s = (rowsb0, rowsb1)
        eid_s = (eid0, eid1)
        sidx2_s = (sidx20, sidx21)
        sema_s = (sema0, sema1)
        semb_s = (semb0, semb1)
        semo_s = (semo0, semo1)
        semw_s = (semw0, semw1)
        semc_s = (semc0, semc1)
        cid = lax.axis_index("c")
        sid = lax.axis_index("s")
        wid = sid * _NC + cid

        @pl.when(sid == 0)
        def _init_cnt():
            pltpu.sync_copy(zeros_hbm, cnt_sh)

        for i in range(_K // 16):
            sl = pl.ds(i * 16, 16)
            ones_v[sl] = jnp.ones((16,), jnp.float32)
            neg_v[sl] = jnp.full((16,), -1, jnp.int32)

        def _fill_win(j, c):
            pltpu.sync_copy(neg_v, win_sh.at[pl.ds(sid * N + j * _K, _K)])
            return c
        lax.fori_loop(0, N // _K, _fill_win, 0)
        if N % _K > 0:
            pltpu.sync_copy(neg_v.at[pl.ds(0, N % _K)],
                            win_sh.at[pl.ds(sid * N + (N // _K) * _K, N % _K)])
        plsc.subcore_barrier()

        n_i = (NCH - wid + _NW - 1) // _NW
        cbase = lambda t: (wid + t * _NW) * _K

        @pl.when(n_i > 0)
        def _prologue():
            pltpu.sync_copy(send_hbm.at[pl.ds(cbase(0), _K)], sidx0)
            pltpu.sync_copy(rec_hbm.at[pl.ds(cbase(0), _K)], ridx0)
            pltpu.async_copy(linh_hbm.at[sidx0], rowsa0, sema0)
            pltpu.async_copy(linh_hbm.at[ridx0], rowsb0, semb0)

        def _pair(j, c):
            for b in (0, 1):
                t = 2 * j + b
                bn = 1 - b

                @pl.when(t < n_i)
                def _step():
                    base = cbase(t)

                    @pl.when(t + 1 < n_i)
                    def _prefetch():
                        nbase = cbase(t + 1)

                        @pl.when(t + 1 >= 2)
                        def _drain_prev():
                            pltpu.make_async_copy(
                                rowsa_s[bn], g_hbm.at[pl.ds(base, _K)],
                                semo_s[bn]).wait()
                        pltpu.sync_copy(send_hbm.at[pl.ds(nbase, _K)],
                                        sidx_s[bn])
                        pltpu.sync_copy(rec_hbm.at[pl.ds(nbase, _K)],
                                        ridx_s[bn])
                        pltpu.async_copy(linh_hbm.at[sidx_s[bn]],
                                         rowsa_s[bn], sema_s[bn])
                        pltpu.async_copy(linh_hbm.at[ridx_s[bn]],
                                         rowsb_s[bn], semb_s[bn])

                    pltpu.make_async_copy(linh_hbm.at[sidx_s[b]],
                                          rowsa_s[b], sema_s[b]).wait()
                    pltpu.make_async_copy(linh_hbm.at[ridx_s[b]],
                                          rowsb_s[b], semb_s[b]).wait()
                    ra = rowsa_s[b]
                    rb = rowsb_s[b]

                    def _radd(r, cc):
                        for rr in range(2):
                            for c8 in range(D // 16):
                                sl = pl.ds(c8 * 16, 16)
                                ra[2 * r + rr, sl] = (ra[2 * r + rr, sl]
                                                      + rb[2 * r + rr, sl])
                        return cc
                    lax.fori_loop(0, _K // 2, _radd, 0)
                    pltpu.async_copy(ra, g_hbm.at[pl.ds(base, _K)],
                                     semo_s[b])
                    for r8 in range(_K // 16):
                        sl = pl.ds(r8 * 16, 16)
                        eid_s[b][sl] = base + r8 * 16 + lax.iota(jnp.int32, 16)
                        sidx2_s[b][sl] = sidx_s[b][sl] + sid * N
                    pltpu.sync_copy(eid_s[b], win_sh.at[sidx2_s[b]])
                    pltpu.sync_copy(ones_v, cnt_sh.at[sidx_s[b]], add=True)
            return c

        lax.fori_loop(0, (n_i + 1) // 2, _pair, 0)
        for b in (0, 1):
            lb = jnp.where((n_i - 1) % 2 == b, n_i - 1, n_i - 2)

            @pl.when(lb >= 0)
            def _drain_tail():
                pltpu.make_async_copy(rowsa_s[b],
                                      g_hbm.at[pl.ds(cbase(0), _K)],
                                      semo_s[b]).wait()

        def _chunk2(i, c):
            base = (wid + i * _NW) * _K
            pltpu.sync_copy(send_hbm.at[pl.ds(base, _K)], sidx0)
            pltpu.async_copy(hpws_hbm.at[sidx0], rowsa0, sema0).wait()
            pltpu.sync_copy(rowsa0, g2_hbm.at[pl.ds(base, _K)])
            return c

        n_i2 = (NCH2 - wid + _NW - 1) // _NW
        lax.fori_loop(0, n_i2, _chunk2, 0)

        if REM2 > 0:
            @pl.when(wid == _NW - 1)
            def _tail():
                tb = NCH2 * _K
                pltpu.sync_copy(send_hbm.at[pl.ds(tb, REM2)],
                                sidx0.at[pl.ds(0, REM2)])
                pltpu.async_copy(hpws_hbm.at[sidx0.at[pl.ds(0, REM2)]],
                                 rowsa0.at[pl.ds(0, REM2)], sema0).wait()
                pltpu.sync_copy(rowsa0.at[pl.ds(0, REM2)],
                                g2_hbm.at[pl.ds(tb, REM2)])

        pltpu.sync_copy(win_sh.at[pl.ds(sid * N, N)], win_v)
        pltpu.sync_copy(win_v, win_hbm.at[wid])
        plsc.subcore_barrier()

        @pl.when(sid == 0)
        def _write_cnt():
            pltpu.sync_copy(cnt_sh, cnt_hbm.at[cid])

    return k


def _sc_segsum_make(N, D, E):
    NCH = E // _K
    ZR = 125
    mesh = plsc.VectorSubcoreMesh(core_axis_name="c", subcore_axis_name="s")

    @functools.partial(
        pl.kernel,
        out_type=jax.ShapeDtypeStruct((_NC, N, D), jnp.float32),
        mesh=mesh,
        scratch_types=[
            pltpu.VMEM((_K,), jnp.int32),
            pltpu.VMEM((_K,), jnp.int32),
            pltpu.VMEM((_K, D), jnp.float32),
            pltpu.VMEM((_K, D), jnp.float32),
            pltpu.VMEM((ZR, D), jnp.float32),
            pltpu.VMEM_SHARED((N, D), jnp.float32),
            pltpu.SemaphoreType.DMA,
            pltpu.SemaphoreType.DMA,
            pltpu.SemaphoreType.DMA,
            pltpu.SemaphoreType.DMA,
        ],
    )
    def k(etanew_hbm, rec_hbm, w_hbm, ridx0, ridx1, env0, env1, zv, w_sh,
          seme0, seme1, sems0, sems1):
        ridx_s = (ridx0, ridx1)
        env_s = (env0, env1)
        seme_s = (seme0, seme1)
        sems_s = (sems0, sems1)
        cid = lax.axis_index("c")
        sid = lax.axis_index("s")
        wid = sid * _NC + cid

        def _zrow(r, c):
            for c8 in range(D // 16):
                zv[r, pl.ds(c8 * 16, 16)] = jnp.zeros((16,), jnp.float32)
            return c
        lax.fori_loop(0, ZR, _zrow, 0)
        nstripe = N // (_NS * ZR)
        def _zcp(j, c):
            pltpu.sync_copy(zv, w_sh.at[pl.ds((sid * nstripe + j) * ZR, ZR)])
            return c
        lax.fori_loop(0, nstripe, _zcp, 0)
        plsc.subcore_barrier()

        n_i = (NCH - wid + _NW - 1) // _NW
        cbase = lambda t: (wid + t * _NW) * _K

        @pl.when(n_i > 0)
        def _prologue():
            pltpu.sync_copy(rec_hbm.at[pl.ds(cbase(0), _K)], ridx0)
            pltpu.async_copy(etanew_hbm.at[pl.ds(cbase(0), _K)], env0, seme0)

        def _pair(j, c):
            for b in (0, 1):
                t = 2 * j + b
                bn = 1 - b

                @pl.when(t < n_i)
                def _step():
                    @pl.when(t + 1 < n_i)
                    def _prefetch():
                        nbase = cbase(t + 1)

                        @pl.when(t + 1 >= 2)
                        def _drain_prev():
                            pltpu.make_async_copy(
                                env_s[bn], w_sh.at[ridx_s[bn]],
                                sems_s[bn]).wait()
                        pltpu.sync_copy(rec_hbm.at[pl.ds(nbase, _K)],
                                        ridx_s[bn])
                        pltpu.async_copy(etanew_hbm.at[pl.ds(nbase, _K)],
                                         env_s[bn], seme_s[bn])

                    pltpu.make_async_copy(etanew_hbm.at[pl.ds(cbase(t), _K)],
                                          env_s[b], seme_s[b]).wait()
                    pltpu.async_copy(env_s[b], w_sh.at[ridx_s[b]],
                                     sems_s[b], add=True)
            return c

        lax.fori_loop(0, (n_i + 1) // 2, _pair, 0)
        for b in (0, 1):
            lb = jnp.where((n_i - 1) % 2 == b, n_i - 1, n_i - 2)

            @pl.when(lb >= 0)
            def _drain_tail():
                pltpu.make_async_copy(env_s[b], w_sh.at[ridx_s[b]],
                                      sems_s[b]).wait()
        plsc.subcore_barrier()

        @pl.when(sid == 0)
        def _write():
            pltpu.sync_copy(w_sh, w_hbm.at[cid])

    return k


def _sc_gather2_make(N, D):
    NCH2 = N // _K
    REM2 = N - NCH2 * _K
    mesh = plsc.VectorSubcoreMesh(core_axis_name="c", subcore_axis_name="s")
    sd = jax.ShapeDtypeStruct

    @functools.partial(
        pl.kernel,
        out_type=[sd((N, D), jnp.float32), sd((N, D), jnp.float32)],
        mesh=mesh,
        scratch_types=[
            pltpu.VMEM((_K,), jnp.int32),
            pltpu.VMEM((_K,), jnp.int32),
            pltpu.VMEM((_K, D), jnp.float32),
            pltpu.VMEM((_K, D), jnp.float32),
            pltpu.VMEM((_K, D), jnp.float32),
            pltpu.VMEM((_K, D), jnp.float32),
            pltpu.SemaphoreType.DMA,
            pltpu.SemaphoreType.DMA,
            pltpu.SemaphoreType.DMA,
            pltpu.SemaphoreType.DMA,
        ],
    )
    def k(seg1_hbm, seg2_hbm, wc_hbm, s1g_hbm, s2g_hbm,
          idx0, idx1, r10, r11, r20, r21, semg0, semg1, semo0, semo1):
        idx_s = (idx0, idx1)
        r1_s = (r10, r11)
        r2_s = (r20, r21)
        semg_s = (semg0, semg1)
        semo_s = (semo0, semo1)
        cid = lax.axis_index("c")
        sid = lax.axis_index("s")
        wid = sid * _NC + cid

        n_i = (NCH2 - wid + _NW - 1) // _NW
        cbase = lambda t: (wid + t * _NW) * _K

        @pl.when(n_i > 0)
        def _prologue():
            pltpu.sync_copy(wc_hbm.at[pl.ds(cbase(0), _K)], idx0)
            pltpu.async_copy(seg1_hbm.at[idx0], r10, semg0)
            pltpu.async_copy(seg2_hbm.at[idx0], r20, semg0)

        def _pair(j, c):
            for b in (0, 1):
                t = 2 * j + b
                bn = 1 - b

                @pl.when(t < n_i)
                def _step():
                    base = cbase(t)

                    @pl.when(t + 1 < n_i)
                    def _prefetch():
                        nbase = cbase(t + 1)

                        @pl.when(t + 1 >= 2)
                        def _drain_prev():
                            pltpu.make_async_copy(
                                r1_s[bn], s1g_hbm.at[pl.ds(base, _K)],
                                semo_s[bn]).wait()
                            pltpu.make_async_copy(
                                r2_s[bn], s2g_hbm.at[pl.ds(base, _K)],
                                semo_s[bn]).wait()
                        pltpu.sync_copy(wc_hbm.at[pl.ds(nbase, _K)],
                                        idx_s[bn])
                        pltpu.async_copy(seg1_hbm.at[idx_s[bn]],
                                         r1_s[bn], semg_s[bn])
                        pltpu.async_copy(seg2_hbm.at[idx_s[bn]],
                                         r2_s[bn], semg_s[bn])

                    pltpu.make_async_copy(seg1_hbm.at[idx_s[b]],
                                          r1_s[b], semg_s[b]).wait()
                    pltpu.make_async_copy(seg2_hbm.at[idx_s[b]],
                                          r2_s[b], semg_s[b]).wait()
                    pltpu.async_copy(r1_s[b], s1g_hbm.at[pl.ds(base, _K)],
                                     semo_s[b])
                    pltpu.async_copy(r2_s[b], s2g_hbm.at[pl.ds(base, _K)],
                                     semo_s[b])
            return c

        lax.fori_loop(0, (n_i + 1) // 2, _pair, 0)
        for b in (0, 1):
            lb = jnp.where((n_i - 1) % 2 == b, n_i - 1, n_i - 2)

            @pl.when(lb >= 0)
            def _drain_tail():
                pltpu.make_async_copy(r1_s[b], s1g_hbm.at[pl.ds(cbase(0), _K)],
                                      semo_s[b]).wait()
                pltpu.make_async_copy(r2_s[b], s2g_hbm.at[pl.ds(cbase(0), _K)],
                                      semo_s[b]).wait()

        if REM2 > 0:
            @pl.when(wid == _NW - 1)
            def _tail():
                tb = NCH2 * _K
                pltpu.sync_copy(wc_hbm.at[pl.ds(tb, REM2)],
                                idx0.at[pl.ds(0, REM2)])
                cp_a = pltpu.async_copy(seg1_hbm.at[idx0.at[pl.ds(0, REM2)]],
                                        r10.at[pl.ds(0, REM2)], semg0)
                cp_b = pltpu.async_copy(seg2_hbm.at[idx0.at[pl.ds(0, REM2)]],
                                        r20.at[pl.ds(0, REM2)], semg0)
                cp_a.wait()
                cp_b.wait()
                pltpu.sync_copy(r10.at[pl.ds(0, REM2)],
                                s1g_hbm.at[pl.ds(tb, REM2)])
                pltpu.sync_copy(r20.at[pl.ds(0, REM2)],
                                s2g_hbm.at[pl.ds(tb, REM2)])

    return k


def _combine_body(winp_ref, cntp_ref, win_ref, cnt_ref):
    win_ref[...] = jnp.max(winp_ref[...], axis=0, keepdims=True)
    cnt_ref[...] = jnp.sum(cntp_ref[...], axis=0, keepdims=True)


def _combine(win_parts, cnt_parts):
    N = win_parts.shape[1]
    full = lambda shape: pl.BlockSpec(shape, lambda: (0, 0))
    return pl.pallas_call(
        _combine_body,
        in_specs=[full((_NW, N)), full((_NC, N))],
        out_specs=[full((1, N)), full((1, N))],
        out_shape=[jax.ShapeDtypeStruct((1, N), jnp.int32),
                   jax.ShapeDtypeStruct((1, N), jnp.float32)],
    )(win_parts, cnt_parts)


def _node_mm_body(h_ref, p_ref, Wlin_ref, blin_ref, Ws_ref, bs_ref, Wr_ref,
                  br_ref, Wp1_ref, bp1_ref, Wp2_ref, bp2_ref,
                  linh_ref, hpWs_ref, hpWr_ref, pWp1_ref, pWp2_ref):
    hb = h_ref[...]
    pb = p_ref[...]
    hp = jnp.concatenate([hb, pb], axis=1)
    f32 = jnp.float32
    linh_ref[...] = jnp.dot(hb, Wlin_ref[...], preferred_element_type=f32) + blin_ref[...]
    hpWs_ref[...] = jnp.dot(hp, Ws_ref[...], preferred_element_type=f32) + bs_ref[...]
    hpWr_ref[...] = jnp.dot(hp, Wr_ref[...], preferred_element_type=f32) + br_ref[...]
    pWp1_ref[...] = jnp.dot(pb, Wp1_ref[...], preferred_element_type=f32) + bp1_ref[...]
    pWp2_ref[...] = jnp.dot(pb, Wp2_ref[...], preferred_element_type=f32) + bp2_ref[...]


def _node_matmuls(h, p, W_lin, b_lin, Ws, bs, Wr, br, Wp1, bp1, Wp2, bp2):
    N, D = h.shape
    R = 1000
    grid = (N // R,)
    row = pl.BlockSpec((R, D), lambda i: (i, 0))
    row2 = pl.BlockSpec((R, 2 * D), lambda i: (i, 0))
    wfull = lambda shape: pl.BlockSpec(shape, lambda i: (0, 0))
    out_sd = jax.ShapeDtypeStruct((N, D), jnp.float32)
    return pl.pallas_call(
        _node_mm_body,
        grid=grid,
        in_specs=[row, row,
                  wfull((D, D)), wfull((1, D)),
                  wfull((2 * D, D)), wfull((1, D)),
                  wfull((2 * D, D)), wfull((1, D)),
                  wfull((D, D)), wfull((1, D)),
                  wfull((D, D)), wfull((1, D))],
        out_specs=[row, row, row, row, row],
        out_shape=[out_sd] * 5,
    )(h, p, W_lin, b_lin.reshape(1, D), Ws, bs.reshape(1, D),
      Wr, br.reshape(1, D), Wp1, bp1.reshape(1, D), Wp2, bp2.reshape(1, D))


def _edge_eta_body(g_ref, e_ref, Wlin_ref, b3_ref,
                   etanew_ref, s_ref, sum_ref, sumsq_ref):
    x = g_ref[...] + jnp.dot(e_ref[...], Wlin_ref[...],
                             preferred_element_type=jnp.float32) + b3_ref[...]
    eta = jax.nn.sigmoid(x)
    s = jnp.sum(eta, axis=1, keepdims=True)
    etanew_ref[...] = eta / s
    s_ref[...] = s
    bsum = jnp.sum(eta, axis=0, keepdims=True)
    bsq = jnp.sum(eta * eta, axis=0, keepdims=True)

    @pl.when(pl.program_id(0) == 0)
    def _init():
        sum_ref[...] = bsum
        sumsq_ref[...] = bsq

    @pl.when(pl.program_id(0) != 0)
    def _acc():
        sum_ref[...] += bsum
        sumsq_ref[...] += bsq


def _edge_eta(g, e, W_lin, b_lin):
    E, D = e.shape
    BE = 1000
    grid = (E // BE,)
    row = pl.BlockSpec((BE, D), lambda i: (i, 0))
    col = pl.BlockSpec((BE, 1), lambda i: (i, 0))
    wfull = lambda shape: pl.BlockSpec(shape, lambda i: (0, 0))
    return pl.pallas_call(
        _edge_eta_body,
        grid=grid,
        in_specs=[row, row, wfull((D, D)), wfull((1, D))],
        out_specs=[row, col, wfull((1, D)), wfull((1, D))],
        out_shape=[jax.ShapeDtypeStruct((E, D), jnp.float32),
                   jax.ShapeDtypeStruct((E, 1), jnp.float32),
                   jax.ShapeDtypeStruct((1, D), jnp.float32),
                   jax.ShapeDtypeStruct((1, D), jnp.float32)],
    )(g, e, W_lin, b_lin.reshape(1, D))


def _stats_body(W0_ref, W1_ref, cnt_ref, win_ref, g2_ref, hpWs_ref,
                hpWr_ref, pWp2_ref, sume_ref, sumsqe_ref, n_edges_ref,
                seg1_ref, seg2_ref, mean1_ref, inv1_ref,
                meane_ref, inve_ref, wc_ref):
    E = n_edges_ref[0]
    Nn = n_edges_ref[1]
    Ef = E.astype(jnp.float32)
    W = W0_ref[...] + W1_ref[...]
    wc_ref[...] = jnp.clip(win_ref[...], 0, Nn - 1)
    seg1 = hpWr_ref[...] * W
    seg2 = pWp2_ref[...] * W
    seg1_ref[...] = seg1
    seg2_ref[...] = seg2
    cnt = cnt_ref[...]
    hpWs = hpWs_ref[...]
    g2 = g2_ref[...]
    A1 = jnp.sum(cnt * hpWs, axis=0, keepdims=True)
    B1 = jnp.sum(cnt * hpWs * hpWs, axis=0, keepdims=True)
    S1 = jnp.sum(seg1, axis=0, keepdims=True)
    C1 = jnp.sum(2.0 * g2 * seg1 + seg1 * seg1, axis=0, keepdims=True)
    part1 = A1 + S1
    part2 = B1 + C1

    @pl.when(pl.program_id(0) == 0)
    def _init():
        mean1_ref[...] = part1
        inv1_ref[...] = part2

    @pl.when(pl.program_id(0) != 0)
    def _acc():
        mean1_ref[...] += part1
        inv1_ref[...] += part2

    @pl.when(pl.program_id(0) == pl.num_programs(0) - 1)
    def _fin():
        mean1 = mean1_ref[...] / Ef
        var1 = inv1_ref[...] / Ef - mean1 * mean1
        mean1_ref[...] = mean1
        inv1_ref[...] = lax.rsqrt(var1 + _EPS)
        meane = sume_ref[...] / Ef
        vare = sumsqe_ref[...] / Ef - meane * meane
        meane_ref[...] = meane
        inve_ref[...] = lax.rsqrt(vare + _EPS)


def _stats(W0, W1, cnt, win, g2, hpWs, hpWr, pWp2, sum_eta, sumsq_eta, E):
    N, D = W0.shape
    R = 2000
    grid = (N // R,)
    row = pl.BlockSpec((R, D), lambda i: (i, 0))
    col = pl.BlockSpec((R, 1), lambda i: (i, 0))
    bc = pl.BlockSpec((1, D), lambda i: (0, 0))
    sd = jax.ShapeDtypeStruct
    return pl.pallas_call(
        _stats_body,
        grid=grid,
        in_specs=[row, row, col, col, row, row, row, row, bc, bc,
                  pl.BlockSpec(memory_space=pltpu.SMEM)],
        out_specs=[row, row, bc, bc, bc, bc, col],
        out_shape=[sd((N, D), jnp.float32), sd((N, D), jnp.float32),
                   sd((1, D), jnp.float32), sd((1, D), jnp.float32),
                   sd((1, D), jnp.float32), sd((1, D), jnp.float32),
                   sd((N, 1), jnp.int32)],
    )(W0, W1, cnt.reshape(N, 1), win.reshape(N, 1), g2, hpWs, hpWr, pWp2,
      sum_eta, sumsq_eta, jnp.array([E, N], dtype=jnp.int32))


def _node_out_body(h_ref, p_ref, hpWs_ref, pWp1_ref, s1g_ref, s2g_ref,
                   win_ref, mean1_ref, inv1_ref, gamma_ref, beta_ref,
                   n_ref, hout_ref, pout_ref):
    Nn = n_ref[0]
    win = win_ref[...]
    has = win >= 0
    use = jnp.logical_and(has, win < Nn)
    h = h_ref[...]
    p = p_ref[...]
    x1 = hpWs_ref[...] + jnp.where(use, s1g_ref[...], 0.0)
    bn1 = (x1 - mean1_ref[...]) * inv1_ref[...] * gamma_ref[...] + beta_ref[...]
    hn = h + jnp.maximum(bn1, 0.0)
    hout_ref[...] = jnp.where(has, hn, h)
    x2 = pWp1_ref[...] + jnp.where(use, s2g_ref[...], 0.0)
    pout_ref[...] = jnp.where(has, p + jnp.tanh(x2), p)


def _node_out(h, p, hpWs, pWp1, s1g, s2g, win, mean1, inv1, gamma, beta, N_dim):
    N, D = h.shape
    R = 1000
    grid = (N // R,)
    row = pl.BlockSpec((R, D), lambda i: (i, 0))
    col = pl.BlockSpec((R, 1), lambda i: (i, 0))
    bc = lambda shape: pl.BlockSpec(shape, lambda i: (0, 0))
    sd = jax.ShapeDtypeStruct
    return pl.pallas_call(
        _node_out_body,
        grid=grid,
        in_specs=[row, row, row, row, row, row, col,
                  bc((1, D)), bc((1, D)), bc((1, D)), bc((1, D)),
                  pl.BlockSpec(memory_space=pltpu.SMEM)],
        out_specs=[row, row],
        out_shape=[sd((N, D), jnp.float32), sd((N, D), jnp.float32)],
    )(h, p, hpWs, pWp1, s1g, s2g, win.reshape(N, 1), mean1, inv1,
      gamma.reshape(1, D), beta.reshape(1, D),
      jnp.array([N_dim], dtype=jnp.int32))


def _e_out_body(e_ref, en_ref, s_ref, meane_ref, inve_ref, gamma_ref,
                beta_ref, eout_ref):
    eta = en_ref[...] * s_ref[...]
    bn = (eta - meane_ref[...]) * inve_ref[...] * gamma_ref[...] + beta_ref[...]
    eout_ref[...] = e_ref[...] + jnp.maximum(bn, 0.0)


def _e_out(e, eta_new, s, meane, inve, gamma, beta):
    E, D = e.shape
    BE = 1000
    grid = (E // BE,)
    row = pl.BlockSpec((BE, D), lambda i: (i, 0))
    col = pl.BlockSpec((BE, 1), lambda i: (i, 0))
    bc = lambda shape: pl.BlockSpec(shape, lambda i: (0, 0))
    return pl.pallas_call(
        _e_out_body,
        grid=grid,
        in_specs=[row, row, col, bc((1, D)), bc((1, D)), bc((1, D)), bc((1, D))],
        out_specs=row,
        out_shape=jax.ShapeDtypeStruct((E, D), jnp.float32),
    )(e, eta_new, s, meane, inve, gamma.reshape(1, D), beta.reshape(1, D))


def kernel(h, e, p, edge_index, W_lin, b_lin, Ws, bs, Wr, br, Wp1, bp1,
           Wp2, bp2, gamma, beta):
    N, D = h.shape
    E = e.shape[0]
    send = edge_index[0]
    rec = edge_index[1]

    lin_h, hpWs, hpWr, pWp1, pWp2 = _node_matmuls(
        h, p, W_lin, b_lin, Ws, bs, Wr, br, Wp1, bp1, Wp2, bp2)

    zeros_n = jnp.zeros((N,), jnp.float32)
    g, g2, win_parts, cnt_parts = _sc_gather_make(N, D, E)(
        lin_h, hpWs, send, rec, zeros_n)
    win2, cnt2 = _combine(win_parts, cnt_parts)
    win = win2.reshape(N)
    cnt = cnt2.reshape(N)

    eta_new, s, sum_eta, sumsq_eta = _edge_eta(g, e, W_lin, b_lin)

    W_parts = _sc_segsum_make(N, D, E)(eta_new, rec)

    seg1, seg2, mean1, inv1, meane, inve, wc = _stats(
        W_parts[0], W_parts[1], cnt, win, g2, hpWs, hpWr, pWp2,
        sum_eta, sumsq_eta, E)

    s1g, s2g = _sc_gather2_make(N, D)(seg1, seg2, wc.reshape(N))

    h_out, p_out = _node_out(h, p, hpWs, pWp1, s1g, s2g, win, mean1, inv1,
                             gamma, beta, N)
    e_out = _e_out(e, eta_new, s, meane, inve, gamma, beta)
    return (h_out, e_out, p_out)


# speedup vs baseline: 1.0056x; 1.0038x over previous
import functools

import jax
import jax.numpy as jnp
from jax import lax
from jax.experimental import pallas as pl
from jax.experimental.pallas import tpu as pltpu
from jax.experimental.pallas import tpu_sc as plsc

_EPS = 1e-5
_NC, _NS = 2, 16
_NW = _NC * _NS
_K = 128


def _sc_gather_make(N, D, E):
    NCH = E // _K
    NCH2 = N // _K
    REM2 = N - NCH2 * _K
    mesh = plsc.VectorSubcoreMesh(core_axis_name="c", subcore_axis_name="s")
    sd = jax.ShapeDtypeStruct

    @functools.partial(
        pl.kernel,
        out_type=[sd((E, D), jnp.float32),
                  sd((N, D), jnp.float32),
                  sd((_NW, N), jnp.int32),
                  sd((_NC, N), jnp.float32)],
        mesh=mesh,
    scratch_types=[
            pltpu.VMEM((_K,), jnp.int32),
            pltpu.VMEM((_K,), jnp.int32),
            pltpu.VMEM((_K,), jnp.int32),
            pltpu.VMEM((_K,), jnp.int32),
            pltpu.VMEM((_K, D), jnp.float32),
            pltpu.VMEM((_K, D), jnp.float32),
            pltpu.VMEM((_K, D), jnp.float32),
            pltpu.VMEM((_K, D), jnp.float32),
            pltpu.VMEM((_K,), jnp.int32),
            pltpu.VMEM((_K,), jnp.int32),
            pltpu.VMEM((_K,), jnp.int32),
            pltpu.VMEM((_K,), jnp.int32),
            pltpu.VMEM((_K,), jnp.int32),
            pltpu.VMEM((_K,), jnp.float32),
            pltpu.VMEM((N,), jnp.int32),
            pltpu.VMEM_SHARED((_NS * N,), jnp.int32),
            pltpu.VMEM_SHARED((N,), jnp.float32),
            pltpu.SemaphoreType.DMA,
            pltpu.SemaphoreType.DMA,
            pltpu.SemaphoreType.DMA,
            pltpu.SemaphoreType.DMA,
            pltpu.SemaphoreType.DMA,
            pltpu.SemaphoreType.DMA,
        ],
    )
    def k(linh_hbm, hpws_hbm, send_hbm, rec_hbm, zeros_hbm,
          g_hbm, g2_hbm, win_hbm, cnt_hbm,
          sidx0, sidx1, ridx0, ridx1, rowsa0, rowsa1, rowsb0, rowsb1,
          eid0, eid1, sidx20, sidx21, neg_v, ones_v,
          win_v, win_sh, cnt_sh, sema0, sema1, semb0, semb1, semo0, semo1):
        sidx_s = (sidx0, sidx1)
        ridx_s = (ridx0, ridx1)
        rowsa_s = (rowsa0, rowsa1)
        rowsb_s = (rowsb0, rowsb1)
        eid_s = (eid0, eid1)
        sidx2_s = (sidx20, sidx21)
        sema_s = (sema0, sema1)
        semb_s = (semb0, semb1)
        semo_s = (semo0, semo1)
        cid = lax.axis_index("c")
        sid = lax.axis_index("s")
        wid = sid * _NC + cid

        @pl.when(sid == 0)
        def _init_cnt():
            pltpu.sync_copy(zeros_hbm, cnt_sh)

        for i in range(_K // 16):
            sl = pl.ds(i * 16, 16)
            ones_v[sl] = jnp.ones((16,), jnp.float32)
            neg_v[sl] = jnp.full((16,), -1, jnp.int32)

        def _fill_win(j, c):
            pltpu.sync_copy(neg_v, win_sh.at[pl.ds(sid * N + j * _K, _K)])
            return c
        lax.fori_loop(0, N // _K, _fill_win, 0)
        if N % _K > 0:
            pltpu.sync_copy(neg_v.at[pl.ds(0, N % _K)],
                            win_sh.at[pl.ds(sid * N + (N // _K) * _K, N % _K)])
        plsc.subcore_barrier()

        n_i = (NCH - wid + _NW - 1) // _NW
        cbase = lambda t: (wid + t * _NW) * _K

        @pl.when(n_i > 0)
        def _prologue():
            pltpu.sync_copy(send_hbm.at[pl.ds(cbase(0), _K)], sidx0)
            pltpu.sync_copy(rec_hbm.at[pl.ds(cbase(0), _K)], ridx0)
            pltpu.async_copy(linh_hbm.at[sidx0], rowsa0, sema0)
            pltpu.async_copy(linh_hbm.at[ridx0], rowsb0, semb0)

        def _pair(j, c):
            for b in (0, 1):
                t = 2 * j + b
                bn = 1 - b

                @pl.when(t < n_i)
                def _step():
                    base = cbase(t)

                    @pl.when(t + 1 < n_i)
                    def _prefetch():
                        nbase = cbase(t + 1)
                        pltpu.sync_copy(send_hbm.at[pl.ds(nbase, _K)],
                                        sidx_s[bn])
                        pltpu.sync_copy(rec_hbm.at[pl.ds(nbase, _K)],
                                        ridx_s[bn])

                        @pl.when(t + 1 >= 2)
                        def _drain_prev():
                            pltpu.make_async_copy(
                                rowsa_s[bn], g_hbm.at[pl.ds(base, _K)],
                                semo_s[bn]).wait()
                        pltpu.async_copy(linh_hbm.at[sidx_s[bn]],
                                         rowsa_s[bn], sema_s[bn])
                        pltpu.async_copy(linh_hbm.at[ridx_s[bn]],
                                         rowsb_s[bn], semb_s[bn])

                    pltpu.make_async_copy(linh_hbm.at[sidx_s[b]],
                                          rowsa_s[b], sema_s[b]).wait()
                    pltpu.make_async_copy(linh_hbm.at[ridx_s[b]],
                                          rowsb_s[b], semb_s[b]).wait()
                    ra = rowsa_s[b]
                    rb = rowsb_s[b]

                    def _radd(r, cc):
                        for rr in range(2):
                            for c8 in range(D // 16):
                                sl = pl.ds(c8 * 16, 16)
                                ra[2 * r + rr, sl] = (ra[2 * r + rr, sl]
                                                      + rb[2 * r + rr, sl])
                        return cc
                    lax.fori_loop(0, _K // 2, _radd, 0)
                    pltpu.async_copy(ra, g_hbm.at[pl.ds(base, _K)],
                                     semo_s[b])
                    for r8 in range(_K // 16):
                        sl = pl.ds(r8 * 16, 16)
                        eid_s[b][sl] = base + r8 * 16 + lax.iota(jnp.int32, 16)
                        sidx2_s[b][sl] = sidx_s[b][sl] + sid * N
                    pltpu.sync_copy(eid_s[b], win_sh.at[sidx2_s[b]])
                    pltpu.sync_copy(ones_v, cnt_sh.at[sidx_s[b]], add=True)
            return c

        lax.fori_loop(0, (n_i + 1) // 2, _pair, 0)
        for b in (0, 1):
            lb = jnp.where((n_i - 1) % 2 == b, n_i - 1, n_i - 2)

            @pl.when(lb >= 0)
            def _drain_tail():
                pltpu.make_async_copy(rowsa_s[b],
                                      g_hbm.at[pl.ds(cbase(0), _K)],
                                      semo_s[b]).wait()

        def _chunk2(i, c):
            base = (wid + i * _NW) * _K
            pltpu.sync_copy(send_hbm.at[pl.ds(base, _K)], sidx0)
            pltpu.async_copy(hpws_hbm.at[sidx0], rowsa0, sema0).wait()
            pltpu.sync_copy(rowsa0, g2_hbm.at[pl.ds(base, _K)])
            return c

        n_i2 = (NCH2 - wid + _NW - 1) // _NW
        lax.fori_loop(0, n_i2, _chunk2, 0)

        if REM2 > 0:
            @pl.when(wid == _NW - 1)
            def _tail():
                tb = NCH2 * _K
                pltpu.sync_copy(send_hbm.at[pl.ds(tb, REM2)],
                                sidx0.at[pl.ds(0, REM2)])
                pltpu.async_copy(hpws_hbm.at[sidx0.at[pl.ds(0, REM2)]],
                                 rowsa0.at[pl.ds(0, REM2)], sema0).wait()
                pltpu.sync_copy(rowsa0.at[pl.ds(0, REM2)],
                                g2_hbm.at[pl.ds(tb, REM2)])

        pltpu.sync_copy(win_sh.at[pl.ds(sid * N, N)], win_v)
        pltpu.sync_copy(win_v, win_hbm.at[wid])
        plsc.subcore_barrier()

        @pl.when(sid == 0)
        def _write_cnt():
            pltpu.sync_copy(cnt_sh, cnt_hbm.at[cid])

    return k


def _sc_segsum_make(N, D, E):
    NCH = E // _K
    ZR = 125
    mesh = plsc.VectorSubcoreMesh(core_axis_name="c", subcore_axis_name="s")

    @functools.partial(
        pl.kernel,
        out_type=jax.ShapeDtypeStruct((_NC, N, D), jnp.float32),
        mesh=mesh,
        scratch_types=[
            pltpu.VMEM((_K,), jnp.int32),
            pltpu.VMEM((_K,), jnp.int32),
            pltpu.VMEM((_K, D), jnp.float32),
            pltpu.VMEM((_K, D), jnp.float32),
            pltpu.VMEM((ZR, D), jnp.float32),
            pltpu.VMEM_SHARED((N, D), jnp.float32),
            pltpu.SemaphoreType.DMA,
            pltpu.SemaphoreType.DMA,
            pltpu.SemaphoreType.DMA,
            pltpu.SemaphoreType.DMA,
        ],
    )
    def k(etanew_hbm, rec_hbm, w_hbm, ridx0, ridx1, env0, env1, zv, w_sh,
          seme0, seme1, sems0, sems1):
        ridx_s = (ridx0, ridx1)
        env_s = (env0, env1)
        seme_s = (seme0, seme1)
        sems_s = (sems0, sems1)
        cid = lax.axis_index("c")
        sid = lax.axis_index("s")
        wid = sid * _NC + cid

        def _zrow(r, c):
            for c8 in range(D // 16):
                zv[r, pl.ds(c8 * 16, 16)] = jnp.zeros((16,), jnp.float32)
            return c
        lax.fori_loop(0, ZR, _zrow, 0)
        nstripe = N // (_NS * ZR)
        def _zcp(j, c):
            pltpu.sync_copy(zv, w_sh.at[pl.ds((sid * nstripe + j) * ZR, ZR)])
            return c
        lax.fori_loop(0, nstripe, _zcp, 0)
        plsc.subcore_barrier()

        n_i = (NCH - wid + _NW - 1) // _NW
        cbase = lambda t: (wid + t * _NW) * _K

        @pl.when(n_i > 0)
        def _prologue():
            pltpu.sync_copy(rec_hbm.at[pl.ds(cbase(0), _K)], ridx0)
            pltpu.async_copy(etanew_hbm.at[pl.ds(cbase(0), _K)], env0, seme0)

        def _pair(j, c):
            for b in (0, 1):
                t = 2 * j + b
                bn = 1 - b

                @pl.when(t < n_i)
                def _step():
                    @pl.when(t + 1 < n_i)
                    def _prefetch():
                        nbase = cbase(t + 1)

                        @pl.when(t + 1 >= 2)
                        def _drain_prev():
                            pltpu.make_async_copy(
                                env_s[bn], w_sh.at[ridx_s[bn]],
                                sems_s[bn]).wait()
                        pltpu.sync_copy(rec_hbm.at[pl.ds(nbase, _K)],
                                        ridx_s[bn])
                        pltpu.async_copy(etanew_hbm.at[pl.ds(nbase, _K)],
                                         env_s[bn], seme_s[bn])

                    pltpu.make_async_copy(etanew_hbm.at[pl.ds(cbase(t), _K)],
                                          env_s[b], seme_s[b]).wait()
                    pltpu.async_copy(env_s[b], w_sh.at[ridx_s[b]],
                                     sems_s[b], add=True)
            return c

        lax.fori_loop(0, (n_i + 1) // 2, _pair, 0)
        for b in (0, 1):
            lb = jnp.where((n_i - 1) % 2 == b, n_i - 1, n_i - 2)

            @pl.when(lb >= 0)
            def _drain_tail():
                pltpu.make_async_copy(env_s[b], w_sh.at[ridx_s[b]],
                                      sems_s[b]).wait()
        plsc.subcore_barrier()

        @pl.when(sid == 0)
        def _write():
            pltpu.sync_copy(w_sh, w_hbm.at[cid])

    return k


def _sc_gather2_make(N, D):
    NCH2 = N // _K
    REM2 = N - NCH2 * _K
    mesh = plsc.VectorSubcoreMesh(core_axis_name="c", subcore_axis_name="s")
    sd = jax.ShapeDtypeStruct

    @functools.partial(
        pl.kernel,
        out_type=[sd((N, D), jnp.float32), sd((N, D), jnp.float32)],
        mesh=mesh,
        scratch_types=[
            pltpu.VMEM((_K,), jnp.int32),
            pltpu.VMEM((_K,), jnp.int32),
            pltpu.VMEM((_K, D), jnp.float32),
            pltpu.VMEM((_K, D), jnp.float32),
            pltpu.VMEM((_K, D), jnp.float32),
            pltpu.VMEM((_K, D), jnp.float32),
            pltpu.SemaphoreType.DMA,
            pltpu.SemaphoreType.DMA,
            pltpu.SemaphoreType.DMA,
            pltpu.SemaphoreType.DMA,
        ],
    )
    def k(seg1_hbm, seg2_hbm, wc_hbm, s1g_hbm, s2g_hbm,
          idx0, idx1, r10, r11, r20, r21, semg0, semg1, semo0, semo1):
        idx_s = (idx0, idx1)
        r1_s = (r10, r11)
        r2_s = (r20, r21)
        semg_s = (semg0, semg1)
        semo_s = (semo0, semo1)
        cid = lax.axis_index("c")
        sid = lax.axis_index("s")
        wid = sid * _NC + cid

        n_i = (NCH2 - wid + _NW - 1) // _NW
        cbase = lambda t: (wid + t * _NW) * _K

        @pl.when(n_i > 0)
        def _prologue():
            pltpu.sync_copy(wc_hbm.at[pl.ds(cbase(0), _K)], idx0)
            pltpu.async_copy(seg1_hbm.at[idx0], r10, semg0)
            pltpu.async_copy(seg2_hbm.at[idx0], r20, semg0)

        def _pair(j, c):
            for b in (0, 1):
                t = 2 * j + b
                bn = 1 - b

                @pl.when(t < n_i)
                def _step():
                    base = cbase(t)

                    @pl.when(t + 1 < n_i)
                    def _prefetch():
                        nbase = cbase(t + 1)

                        @pl.when(t + 1 >= 2)
                        def _drain_prev():
                            pltpu.make_async_copy(
                                r1_s[bn], s1g_hbm.at[pl.ds(base, _K)],
                                semo_s[bn]).wait()
                            pltpu.make_async_copy(
                                r2_s[bn], s2g_hbm.at[pl.ds(base, _K)],
                                semo_s[bn]).wait()
                        pltpu.sync_copy(wc_hbm.at[pl.ds(nbase, _K)],
                                        idx_s[bn])
                        pltpu.async_copy(seg1_hbm.at[idx_s[bn]],
                                         r1_s[bn], semg_s[bn])
                        pltpu.async_copy(seg2_hbm.at[idx_s[bn]],
                                         r2_s[bn], semg_s[bn])

                    pltpu.make_async_copy(seg1_hbm.at[idx_s[b]],
                                          r1_s[b], semg_s[b]).wait()
                    pltpu.make_async_copy(seg2_hbm.at[idx_s[b]],
                                          r2_s[b], semg_s[b]).wait()
                    pltpu.async_copy(r1_s[b], s1g_hbm.at[pl.ds(base, _K)],
                                     semo_s[b])
                    pltpu.async_copy(r2_s[b], s2g_hbm.at[pl.ds(base, _K)],
                                     semo_s[b])
            return c

        lax.fori_loop(0, (n_i + 1) // 2, _pair, 0)
        for b in (0, 1):
            lb = jnp.where((n_i - 1) % 2 == b, n_i - 1, n_i - 2)

            @pl.when(lb >= 0)
            def _drain_tail():
                pltpu.make_async_copy(r1_s[b], s1g_hbm.at[pl.ds(cbase(0), _K)],
                                      semo_s[b]).wait()
                pltpu.make_async_copy(r2_s[b], s2g_hbm.at[pl.ds(cbase(0), _K)],
                                      semo_s[b]).wait()

        if REM2 > 0:
            @pl.when(wid == _NW - 1)
            def _tail():
                tb = NCH2 * _K
                pltpu.sync_copy(wc_hbm.at[pl.ds(tb, REM2)],
                                idx0.at[pl.ds(0, REM2)])
                cp_a = pltpu.async_copy(seg1_hbm.at[idx0.at[pl.ds(0, REM2)]],
                                        r10.at[pl.ds(0, REM2)], semg0)
                cp_b = pltpu.async_copy(seg2_hbm.at[idx0.at[pl.ds(0, REM2)]],
                                        r20.at[pl.ds(0, REM2)], semg0)
                cp_a.wait()
                cp_b.wait()
                pltpu.sync_copy(r10.at[pl.ds(0, REM2)],
                                s1g_hbm.at[pl.ds(tb, REM2)])
                pltpu.sync_copy(r20.at[pl.ds(0, REM2)],
                                s2g_hbm.at[pl.ds(tb, REM2)])

    return k


def _combine_body(winp_ref, cntp_ref, win_ref, cnt_ref):
    win_ref[...] = jnp.max(winp_ref[...], axis=0, keepdims=True)
    cnt_ref[...] = jnp.sum(cntp_ref[...], axis=0, keepdims=True)


def _combine(win_parts, cnt_parts):
    N = win_parts.shape[1]
    full = lambda shape: pl.BlockSpec(shape, lambda: (0, 0))
    return pl.pallas_call(
        _combine_body,
        in_specs=[full((_NW, N)), full((_NC, N))],
        out_specs=[full((1, N)), full((1, N))],
        out_shape=[jax.ShapeDtypeStruct((1, N), jnp.int32),
                   jax.ShapeDtypeStruct((1, N), jnp.float32)],
    )(win_parts, cnt_parts)


def _node_mm_body(h_ref, p_ref, Wlin_ref, blin_ref, Ws_ref, bs_ref, Wr_ref,
                  br_ref, Wp1_ref, bp1_ref, Wp2_ref, bp2_ref,
                  linh_ref, hpWs_ref, hpWr_ref, pWp1_ref, pWp2_ref):
    hb = h_ref[...]
    pb = p_ref[...]
    hp = jnp.concatenate([hb, pb], axis=1)
    f32 = jnp.float32
    linh_ref[...] = jnp.dot(hb, Wlin_ref[...], preferred_element_type=f32) + blin_ref[...]
    hpWs_ref[...] = jnp.dot(hp, Ws_ref[...], preferred_element_type=f32) + bs_ref[...]
    hpWr_ref[...] = jnp.dot(hp, Wr_ref[...], preferred_element_type=f32) + br_ref[...]
    pWp1_ref[...] = jnp.dot(pb, Wp1_ref[...], preferred_element_type=f32) + bp1_ref[...]
    pWp2_ref[...] = jnp.dot(pb, Wp2_ref[...], preferred_element_type=f32) + bp2_ref[...]


def _node_matmuls(h, p, W_lin, b_lin, Ws, bs, Wr, br, Wp1, bp1, Wp2, bp2):
    N, D = h.shape
    R = 1000
    grid = (N // R,)
    row = pl.BlockSpec((R, D), lambda i: (i, 0))
    row2 = pl.BlockSpec((R, 2 * D), lambda i: (i, 0))
    wfull = lambda shape: pl.BlockSpec(shape, lambda i: (0, 0))
    out_sd = jax.ShapeDtypeStruct((N, D), jnp.float32)
    return pl.pallas_call(
        _node_mm_body,
        grid=grid,
        in_specs=[row, row,
                  wfull((D, D)), wfull((1, D)),
                  wfull((2 * D, D)), wfull((1, D)),
                  wfull((2 * D, D)), wfull((1, D)),
                  wfull((D, D)), wfull((1, D)),
                  wfull((D, D)), wfull((1, D))],
        out_specs=[row, row, row, row, row],
        out_shape=[out_sd] * 5,
    )(h, p, W_lin, b_lin.reshape(1, D), Ws, bs.reshape(1, D),
      Wr, br.reshape(1, D), Wp1, bp1.reshape(1, D), Wp2, bp2.reshape(1, D))


def _edge_eta_body(g_ref, e_ref, Wlin_ref, b3_ref,
                   etanew_ref, s_ref, sum_ref, sumsq_ref):
    x = g_ref[...] + jnp.dot(e_ref[...], Wlin_ref[...],
                             preferred_element_type=jnp.float32) + b3_ref[...]
    eta = jax.nn.sigmoid(x)
    s = jnp.sum(eta, axis=1, keepdims=True)
    etanew_ref[...] = eta / s
    s_ref[...] = s
    bsum = jnp.sum(eta, axis=0, keepdims=True)
    bsq = jnp.sum(eta * eta, axis=0, keepdims=True)

    @pl.when(pl.program_id(0) == 0)
    def _init():
        sum_ref[...] = bsum
        sumsq_ref[...] = bsq

    @pl.when(pl.program_id(0) != 0)
    def _acc():
        sum_ref[...] += bsum
        sumsq_ref[...] += bsq


def _edge_eta(g, e, W_lin, b_lin):
    E, D = e.shape
    BE = 1000
    grid = (E // BE,)
    row = pl.BlockSpec((BE, D), lambda i: (i, 0))
    col = pl.BlockSpec((BE, 1), lambda i: (i, 0))
    wfull = lambda shape: pl.BlockSpec(shape, lambda i: (0, 0))
    return pl.pallas_call(
        _edge_eta_body,
        grid=grid,
        in_specs=[row, row, wfull((D, D)), wfull((1, D))],
        out_specs=[row, col, wfull((1, D)), wfull((1, D))],
        out_shape=[jax.ShapeDtypeStruct((E, D), jnp.float32),
                   jax.ShapeDtypeStruct((E, 1), jnp.float32),
                   jax.ShapeDtypeStruct((1, D), jnp.float32),
                   jax.ShapeDtypeStruct((1, D), jnp.float32)],
    )(g, e, W_lin, b_lin.reshape(1, D))


def _stats_body(W0_ref, W1_ref, cnt_ref, win_ref, g2_ref, hpWs_ref,
                hpWr_ref, pWp2_ref, sume_ref, sumsqe_ref, n_edges_ref,
                seg1_ref, seg2_ref, mean1_ref, inv1_ref,
                meane_ref, inve_ref, wc_ref):
    E = n_edges_ref[0]
    Nn = n_edges_ref[1]
    Ef = E.astype(jnp.float32)
    W = W0_ref[...] + W1_ref[...]
    wc_ref[...] = jnp.clip(win_ref[...], 0, Nn - 1)
    seg1 = hpWr_ref[...] * W
    seg2 = pWp2_ref[...] * W
    seg1_ref[...] = seg1
    seg2_ref[...] = seg2
    cnt = cnt_ref[...]
    hpWs = hpWs_ref[...]
    g2 = g2_ref[...]
    A1 = jnp.sum(cnt * hpWs, axis=0, keepdims=True)
    B1 = jnp.sum(cnt * hpWs * hpWs, axis=0, keepdims=True)
    S1 = jnp.sum(seg1, axis=0, keepdims=True)
    C1 = jnp.sum(2.0 * g2 * seg1 + seg1 * seg1, axis=0, keepdims=True)
    part1 = A1 + S1
    part2 = B1 + C1

    @pl.when(pl.program_id(0) == 0)
    def _init():
        mean1_ref[...] = part1
        inv1_ref[...] = part2

    @pl.when(pl.program_id(0) != 0)
    def _acc():
        mean1_ref[...] += part1
        inv1_ref[...] += part2

    @pl.when(pl.program_id(0) == pl.num_programs(0) - 1)
    def _fin():
        mean1 = mean1_ref[...] / Ef
        var1 = inv1_ref[...] / Ef - mean1 * mean1
        mean1_ref[...] = mean1
        inv1_ref[...] = lax.rsqrt(var1 + _EPS)
        meane = sume_ref[...] / Ef
        vare = sumsqe_ref[...] / Ef - meane * meane
        meane_ref[...] = meane
        inve_ref[...] = lax.rsqrt(vare + _EPS)


def _stats(W0, W1, cnt, win, g2, hpWs, hpWr, pWp2, sum_eta, sumsq_eta, E):
    N, D = W0.shape
    R = 2000
    grid = (N // R,)
    row = pl.BlockSpec((R, D), lambda i: (i, 0))
    col = pl.BlockSpec((R, 1), lambda i: (i, 0))
    bc = pl.BlockSpec((1, D), lambda i: (0, 0))
    sd = jax.ShapeDtypeStruct
    return pl.pallas_call(
        _stats_body,
        grid=grid,
        in_specs=[row, row, col, col, row, row, row, row, bc, bc,
                  pl.BlockSpec(memory_space=pltpu.SMEM)],
        out_specs=[row, row, bc, bc, bc, bc, col],
        out_shape=[sd((N, D), jnp.float32), sd((N, D), jnp.float32),
                   sd((1, D), jnp.float32), sd((1, D), jnp.float32),
                   sd((1, D), jnp.float32), sd((1, D), jnp.float32),
                   sd((N, 1), jnp.int32)],
    )(W0, W1, cnt.reshape(N, 1), win.reshape(N, 1), g2, hpWs, hpWr, pWp2,
      sum_eta, sumsq_eta, jnp.array([E, N], dtype=jnp.int32))


def _node_out_body(h_ref, p_ref, hpWs_ref, pWp1_ref, s1g_ref, s2g_ref,
                   win_ref, mean1_ref, inv1_ref, gamma_ref, beta_ref,
                   n_ref, hout_ref, pout_ref):
    Nn = n_ref[0]
    win = win_ref[...]
    has = win >= 0
    use = jnp.logical_and(has, win < Nn)
    h = h_ref[...]
    p = p_ref[...]
    x1 = hpWs_ref[...] + jnp.where(use, s1g_ref[...], 0.0)
    bn1 = (x1 - mean1_ref[...]) * inv1_ref[...] * gamma_ref[...] + beta_ref[...]
    hn = h + jnp.maximum(bn1, 0.0)
    hout_ref[...] = jnp.where(has, hn, h)
    x2 = pWp1_ref[...] + jnp.where(use, s2g_ref[...], 0.0)
    pout_ref[...] = jnp.where(has, p + jnp.tanh(x2), p)


def _node_out(h, p, hpWs, pWp1, s1g, s2g, win, mean1, inv1, gamma, beta, N_dim):
    N, D = h.shape
    R = 1000
    grid = (N // R,)
    row = pl.BlockSpec((R, D), lambda i: (i, 0))
    col = pl.BlockSpec((R, 1), lambda i: (i, 0))
    bc = lambda shape: pl.BlockSpec(shape, lambda i: (0, 0))
    sd = jax.ShapeDtypeStruct
    return pl.pallas_call(
        _node_out_body,
        grid=grid,
        in_specs=[row, row, row, row, row, row, col,
                  bc((1, D)), bc((1, D)), bc((1, D)), bc((1, D)),
                  pl.BlockSpec(memory_space=pltpu.SMEM)],
        out_specs=[row, row],
        out_shape=[sd((N, D), jnp.float32), sd((N, D), jnp.float32)],
    )(h, p, hpWs, pWp1, s1g, s2g, win.reshape(N, 1), mean1, inv1,
      gamma.reshape(1, D), beta.reshape(1, D),
      jnp.array([N_dim], dtype=jnp.int32))


def _e_out_body(e_ref, en_ref, s_ref, meane_ref, inve_ref, gamma_ref,
                beta_ref, eout_ref):
    eta = en_ref[...] * s_ref[...]
    bn = (eta - meane_ref[...]) * inve_ref[...] * gamma_ref[...] + beta_ref[...]
    eout_ref[...] = e_ref[...] + jnp.maximum(bn, 0.0)


def _e_out(e, eta_new, s, meane, inve, gamma, beta):
    E, D = e.shape
    BE = 1000
    grid = (E // BE,)
    row = pl.BlockSpec((BE, D), lambda i: (i, 0))
    col = pl.BlockSpec((BE, 1), lambda i: (i, 0))
    bc = lambda shape: pl.BlockSpec(shape, lambda i: (0, 0))
    return pl.pallas_call(
        _e_out_body,
        grid=grid,
        in_specs=[row, row, col, bc((1, D)), bc((1, D)), bc((1, D)), bc((1, D))],
        out_specs=row,
        out_shape=jax.ShapeDtypeStruct((E, D), jnp.float32),
    )(e, eta_new, s, meane, inve, gamma.reshape(1, D), beta.reshape(1, D))


def kernel(h, e, p, edge_index, W_lin, b_lin, Ws, bs, Wr, br, Wp1, bp1,
           Wp2, bp2, gamma, beta):
    N, D = h.shape
    E = e.shape[0]
    send = edge_index[0]
    rec = edge_index[1]

    lin_h, hpWs, hpWr, pWp1, pWp2 = _node_matmuls(
        h, p, W_lin, b_lin, Ws, bs, Wr, br, Wp1, bp1, Wp2, bp2)

    zeros_n = jnp.zeros((N,), jnp.float32)
    g, g2, win_parts, cnt_parts = _sc_gather_make(N, D, E)(
        lin_h, hpWs, send, rec, zeros_n)
    win2, cnt2 = _combine(win_parts, cnt_parts)
    win = win2.reshape(N)
    cnt = cnt2.reshape(N)

    eta_new, s, sum_eta, sumsq_eta = _edge_eta(g, e, W_lin, b_lin)

    W_parts = _sc_segsum_make(N, D, E)(eta_new, rec)

    seg1, seg2, mean1, inv1, meane, inve, wc = _stats(
        W_parts[0], W_parts[1], cnt, win, g2, hpWs, hpWr, pWp2,
        sum_eta, sumsq_eta, E)

    s1g, s2g = _sc_gather2_make(N, D)(seg1, seg2, wc.reshape(N))

    h_out, p_out = _node_out(h, p, hpWs, pWp1, s1g, s2g, win, mean1, inv1,
                             gamma, beta, N)
    e_out = _e_out(e, eta_new, s, meane, inve, gamma, beta)
    return (h_out, e_out, p_out)


# speedup vs baseline: 1.0093x; 1.0037x over previous
import functools

import jax
import jax.numpy as jnp
from jax import lax
from jax.experimental import pallas as pl
from jax.experimental.pallas import tpu as pltpu
from jax.experimental.pallas import tpu_sc as plsc

_EPS = 1e-5
_NC, _NS = 2, 16
_NW = _NC * _NS
_K = 128


def _sc_gather_make(N, D, E):
    NCH = E // _K
    NCH2 = N // _K
    REM2 = N - NCH2 * _K
    mesh = plsc.VectorSubcoreMesh(core_axis_name="c", subcore_axis_name="s")
    sd = jax.ShapeDtypeStruct

    @functools.partial(
        pl.kernel,
        out_type=[sd((E, D), jnp.float32),
                  sd((N, D), jnp.float32),
                  sd((_NW, N), jnp.int32),
                  sd((_NC, N), jnp.float32)],
        mesh=mesh,
    scratch_types=[
            pltpu.VMEM((_K,), jnp.int32),
            pltpu.VMEM((_K,), jnp.int32),
            pltpu.VMEM((_K,), jnp.int32),
            pltpu.VMEM((_K,), jnp.int32),
            pltpu.VMEM((_K, D), jnp.float32),
            pltpu.VMEM((_K, D), jnp.float32),
            pltpu.VMEM((_K, D), jnp.float32),
            pltpu.VMEM((_K, D), jnp.float32),
            pltpu.VMEM((_K,), jnp.int32),
            pltpu.VMEM((_K,), jnp.int32),
            pltpu.VMEM((_K,), jnp.int32),
            pltpu.VMEM((_K,), jnp.int32),
            pltpu.VMEM((_K,), jnp.int32),
            pltpu.VMEM((_K,), jnp.float32),
            pltpu.VMEM((N,), jnp.int32),
            pltpu.VMEM_SHARED((_NS * N,), jnp.int32),
            pltpu.VMEM_SHARED((N,), jnp.float32),
            pltpu.SemaphoreType.DMA,
            pltpu.SemaphoreType.DMA,
            pltpu.SemaphoreType.DMA,
            pltpu.SemaphoreType.DMA,
            pltpu.SemaphoreType.DMA,
            pltpu.SemaphoreType.DMA,
        ],
    )
    def k(linh_hbm, hpws_hbm, send_hbm, rec_hbm, zeros_hbm,
          g_hbm, g2_hbm, win_hbm, cnt_hbm,
          sidx0, sidx1, ridx0, ridx1, rowsa0, rowsa1, rowsb0, rowsb1,
          eid0, eid1, sidx20, sidx21, neg_v, ones_v,
          win_v, win_sh, cnt_sh, sema0, sema1, semb0, semb1, semo0, semo1):
        sidx_s = (sidx0, sidx1)
        ridx_s = (ridx0, ridx1)
        rowsa_s = (rowsa0, rowsa1)
        rowsb_s = (rowsb0, rowsb1)
        eid_s = (eid0, eid1)
        sidx2_s = (sidx20, sidx21)
        sema_s = (sema0, sema1)
        semb_s = (semb0, semb1)
        semo_s = (semo0, semo1)
        cid = lax.axis_index("c")
        sid = lax.axis_index("s")
        wid = sid * _NC + cid

        @pl.when(sid == 0)
        def _init_cnt():
            pltpu.sync_copy(zeros_hbm, cnt_sh)

        for i in range(_K // 16):
            sl = pl.ds(i * 16, 16)
            ones_v[sl] = jnp.ones((16,), jnp.float32)
            neg_v[sl] = jnp.full((16,), -1, jnp.int32)

        def _fill_win(j, c):
            pltpu.sync_copy(neg_v, win_sh.at[pl.ds(sid * N + j * _K, _K)])
            return c
        lax.fori_loop(0, N // _K, _fill_win, 0)
        if N % _K > 0:
            pltpu.sync_copy(neg_v.at[pl.ds(0, N % _K)],
                            win_sh.at[pl.ds(sid * N + (N // _K) * _K, N % _K)])
        plsc.subcore_barrier()

        n_i = (NCH - wid + _NW - 1) // _NW
        cbase = lambda t: (wid + t * _NW) * _K

        @pl.when(n_i > 0)
        def _prologue():
            pltpu.sync_copy(send_hbm.at[pl.ds(cbase(0), _K)], sidx0)
            pltpu.sync_copy(rec_hbm.at[pl.ds(cbase(0), _K)], ridx0)
            pltpu.async_copy(linh_hbm.at[sidx0], rowsa0, sema0)
            pltpu.async_copy(linh_hbm.at[ridx0], rowsb0, semb0)

        def _pair(j, c):
            for b in (0, 1):
                t = 2 * j + b
                bn = 1 - b

                @pl.when(t < n_i)
                def _step():
                    base = cbase(t)

                    @pl.when(t + 1 < n_i)
                    def _prefetch():
                        nbase = cbase(t + 1)
                        pltpu.sync_copy(send_hbm.at[pl.ds(nbase, _K)],
                                        sidx_s[bn])
                        pltpu.sync_copy(rec_hbm.at[pl.ds(nbase, _K)],
                                        ridx_s[bn])

                        @pl.when(t + 1 >= 2)
                        def _drain_prev():
                            pltpu.make_async_copy(
                                rowsa_s[bn], g_hbm.at[pl.ds(base, _K)],
                                semo_s[bn]).wait()
                        pltpu.async_copy(linh_hbm.at[sidx_s[bn]],
                                         rowsa_s[bn], sema_s[bn])
                        pltpu.async_copy(linh_hbm.at[ridx_s[bn]],
                                         rowsb_s[bn], semb_s[bn])

                    pltpu.make_async_copy(linh_hbm.at[sidx_s[b]],
                                          rowsa_s[b], sema_s[b]).wait()
                    pltpu.make_async_copy(linh_hbm.at[ridx_s[b]],
                                          rowsb_s[b], semb_s[b]).wait()
                    ra = rowsa_s[b]
                    rb = rowsb_s[b]

                    def _radd(r, cc):
                        for rr in range(2):
                            for c8 in range(D // 16):
                                sl = pl.ds(c8 * 16, 16)
                                ra[2 * r + rr, sl] = (ra[2 * r + rr, sl]
                                                      + rb[2 * r + rr, sl])
                        return cc
                    lax.fori_loop(0, _K // 2, _radd, 0)
                    pltpu.async_copy(ra, g_hbm.at[pl.ds(base, _K)],
                                     semo_s[b])
                    for r8 in range(_K // 16):
                        sl = pl.ds(r8 * 16, 16)
                        eid_s[b][sl] = base + r8 * 16 + lax.iota(jnp.int32, 16)
                        sidx2_s[b][sl] = sidx_s[b][sl] + sid * N
                    pltpu.sync_copy(eid_s[b], win_sh.at[sidx2_s[b]])
                    pltpu.sync_copy(ones_v, cnt_sh.at[sidx_s[b]], add=True)
            return c

        lax.fori_loop(0, (n_i + 1) // 2, _pair, 0)
        for b in (0, 1):
            lb = jnp.where((n_i - 1) % 2 == b, n_i - 1, n_i - 2)

            @pl.when(lb >= 0)
            def _drain_tail():
                pltpu.make_async_copy(rowsa_s[b],
                                      g_hbm.at[pl.ds(cbase(0), _K)],
                                      semo_s[b]).wait()

        def _chunk2(i, c):
            base = (wid + i * _NW) * _K
            pltpu.sync_copy(send_hbm.at[pl.ds(base, _K)], sidx0)
            pltpu.async_copy(hpws_hbm.at[sidx0], rowsa0, sema0).wait()
            pltpu.sync_copy(rowsa0, g2_hbm.at[pl.ds(base, _K)])
            return c

        n_i2 = (NCH2 - wid + _NW - 1) // _NW
        lax.fori_loop(0, n_i2, _chunk2, 0)

        if REM2 > 0:
            @pl.when(wid == _NW - 1)
            def _tail():
                tb = NCH2 * _K
                pltpu.sync_copy(send_hbm.at[pl.ds(tb, REM2)],
                                sidx0.at[pl.ds(0, REM2)])
                pltpu.async_copy(hpws_hbm.at[sidx0.at[pl.ds(0, REM2)]],
                                 rowsa0.at[pl.ds(0, REM2)], sema0).wait()
                pltpu.sync_copy(rowsa0.at[pl.ds(0, REM2)],
                                g2_hbm.at[pl.ds(tb, REM2)])

        pltpu.sync_copy(win_sh.at[pl.ds(sid * N, N)], win_v)
        pltpu.sync_copy(win_v, win_hbm.at[wid])
        plsc.subcore_barrier()

        @pl.when(sid == 0)
        def _write_cnt():
            pltpu.sync_copy(cnt_sh, cnt_hbm.at[cid])

    return k


def _sc_segsum_make(N, D, E):
    NCH = E // _K
    ZR = 125
    mesh = plsc.VectorSubcoreMesh(core_axis_name="c", subcore_axis_name="s")

    @functools.partial(
        pl.kernel,
        out_type=jax.ShapeDtypeStruct((_NC, N, D), jnp.float32),
        mesh=mesh,
        scratch_types=[
            pltpu.VMEM((_K,), jnp.int32),
            pltpu.VMEM((_K,), jnp.int32),
            pltpu.VMEM((_K, D), jnp.float32),
            pltpu.VMEM((_K, D), jnp.float32),
            pltpu.VMEM((ZR, D), jnp.float32),
            pltpu.VMEM_SHARED((N, D), jnp.float32),
            pltpu.SemaphoreType.DMA,
            pltpu.SemaphoreType.DMA,
        ],
    )
    def k(etanew_hbm, rec_hbm, w_hbm, ridx0, ridx1, env0, env1, zv, w_sh,
          seme0, seme1):
        ridx_s = (ridx0, ridx1)
        env_s = (env0, env1)
        seme_s = (seme0, seme1)
        cid = lax.axis_index("c")
        sid = lax.axis_index("s")
        wid = sid * _NC + cid

        def _zrow(r, c):
            for c8 in range(D // 16):
                zv[r, pl.ds(c8 * 16, 16)] = jnp.zeros((16,), jnp.float32)
            return c
        lax.fori_loop(0, ZR, _zrow, 0)
        nstripe = N // (_NS * ZR)
        def _zcp(j, c):
            pltpu.sync_copy(zv, w_sh.at[pl.ds((sid * nstripe + j) * ZR, ZR)])
            return c
        lax.fori_loop(0, nstripe, _zcp, 0)
        plsc.subcore_barrier()

        n_i = (NCH - wid + _NW - 1) // _NW
        cbase = lambda t: (wid + t * _NW) * _K

        @pl.when(n_i > 0)
        def _prologue():
            pltpu.sync_copy(rec_hbm.at[pl.ds(cbase(0), _K)], ridx0)
            pltpu.async_copy(etanew_hbm.at[pl.ds(cbase(0), _K)], env0, seme0)

        def _pair(j, c):
            for b in (0, 1):
                t = 2 * j + b
                bn = 1 - b

                @pl.when(t < n_i)
                def _step():
                    @pl.when(t + 1 < n_i)
                    def _prefetch():
                        nbase = cbase(t + 1)
                        pltpu.sync_copy(rec_hbm.at[pl.ds(nbase, _K)],
                                        ridx_s[bn])
                        pltpu.async_copy(etanew_hbm.at[pl.ds(nbase, _K)],
                                         env_s[bn], seme_s[bn])

                    pltpu.make_async_copy(etanew_hbm.at[pl.ds(cbase(t), _K)],
                                          env_s[b], seme_s[b]).wait()
                    pltpu.sync_copy(env_s[b], w_sh.at[ridx_s[b]], add=True)
            return c

        lax.fori_loop(0, (n_i + 1) // 2, _pair, 0)
        plsc.subcore_barrier()

        @pl.when(sid == 0)
        def _write():
            pltpu.sync_copy(w_sh, w_hbm.at[cid])

    return k


def _sc_gather2_make(N, D):
    NCH2 = N // _K
    REM2 = N - NCH2 * _K
    mesh = plsc.VectorSubcoreMesh(core_axis_name="c", subcore_axis_name="s")
    sd = jax.ShapeDtypeStruct

    @functools.partial(
        pl.kernel,
        out_type=[sd((N, D), jnp.float32), sd((N, D), jnp.float32)],
        mesh=mesh,
        scratch_types=[
            pltpu.VMEM((_K,), jnp.int32),
            pltpu.VMEM((_K,), jnp.int32),
            pltpu.VMEM((_K, D), jnp.float32),
            pltpu.VMEM((_K, D), jnp.float32),
            pltpu.VMEM((_K, D), jnp.float32),
            pltpu.VMEM((_K, D), jnp.float32),
            pltpu.SemaphoreType.DMA,
            pltpu.SemaphoreType.DMA,
            pltpu.SemaphoreType.DMA,
            pltpu.SemaphoreType.DMA,
        ],
    )
    def k(seg1_hbm, seg2_hbm, wc_hbm, s1g_hbm, s2g_hbm,
          idx0, idx1, r10, r11, r20, r21, semg0, semg1, semo0, semo1):
        idx_s = (idx0, idx1)
        r1_s = (r10, r11)
        r2_s = (r20, r21)
        semg_s = (semg0, semg1)
        semo_s = (semo0, semo1)
        cid = lax.axis_index("c")
        sid = lax.axis_index("s")
        wid = sid * _NC + cid

        n_i = (NCH2 - wid + _NW - 1) // _NW
        cbase = lambda t: (wid + t * _NW) * _K

        @pl.when(n_i > 0)
        def _prologue():
            pltpu.sync_copy(wc_hbm.at[pl.ds(cbase(0), _K)], idx0)
            pltpu.async_copy(seg1_hbm.at[idx0], r10, semg0)
            pltpu.async_copy(seg2_hbm.at[idx0], r20, semg0)

        def _pair(j, c):
            for b in (0, 1):
                t = 2 * j + b
                bn = 1 - b

                @pl.when(t < n_i)
                def _step():
                    base = cbase(t)

                    @pl.when(t + 1 < n_i)
                    def _prefetch():
                        nbase = cbase(t + 1)

                        @pl.when(t + 1 >= 2)
                        def _drain_prev():
                            pltpu.make_async_copy(
                                r1_s[bn], s1g_hbm.at[pl.ds(base, _K)],
                                semo_s[bn]).wait()
                            pltpu.make_async_copy(
                                r2_s[bn], s2g_hbm.at[pl.ds(base, _K)],
                                semo_s[bn]).wait()
                        pltpu.sync_copy(wc_hbm.at[pl.ds(nbase, _K)],
                                        idx_s[bn])
                        pltpu.async_copy(seg1_hbm.at[idx_s[bn]],
                                         r1_s[bn], semg_s[bn])
                        pltpu.async_copy(seg2_hbm.at[idx_s[bn]],
                                         r2_s[bn], semg_s[bn])

                    pltpu.make_async_copy(seg1_hbm.at[idx_s[b]],
                                          r1_s[b], semg_s[b]).wait()
                    pltpu.make_async_copy(seg2_hbm.at[idx_s[b]],
                                          r2_s[b], semg_s[b]).wait()
                    pltpu.async_copy(r1_s[b], s1g_hbm.at[pl.ds(base, _K)],
                                     semo_s[b])
                    pltpu.async_copy(r2_s[b], s2g_hbm.at[pl.ds(base, _K)],
                                     semo_s[b])
            return c

        lax.fori_loop(0, (n_i + 1) // 2, _pair, 0)
        for b in (0, 1):
            lb = jnp.where((n_i - 1) % 2 == b, n_i - 1, n_i - 2)

            @pl.when(lb >= 0)
            def _drain_tail():
                pltpu.make_async_copy(r1_s[b], s1g_hbm.at[pl.ds(cbase(0), _K)],
                                      semo_s[b]).wait()
                pltpu.make_async_copy(r2_s[b], s2g_hbm.at[pl.ds(cbase(0), _K)],
                                      semo_s[b]).wait()

        if REM2 > 0:
            @pl.when(wid == _NW - 1)
            def _tail():
                tb = NCH2 * _K
                pltpu.sync_copy(wc_hbm.at[pl.ds(tb, REM2)],
                                idx0.at[pl.ds(0, REM2)])
                cp_a = pltpu.async_copy(seg1_hbm.at[idx0.at[pl.ds(0, REM2)]],
                                        r10.at[pl.ds(0, REM2)], semg0)
                cp_b = pltpu.async_copy(seg2_hbm.at[idx0.at[pl.ds(0, REM2)]],
                                        r20.at[pl.ds(0, REM2)], semg0)
                cp_a.wait()
                cp_b.wait()
                pltpu.sync_copy(r10.at[pl.ds(0, REM2)],
                                s1g_hbm.at[pl.ds(tb, REM2)])
                pltpu.sync_copy(r20.at[pl.ds(0, REM2)],
                                s2g_hbm.at[pl.ds(tb, REM2)])

    return k


def _combine_body(winp_ref, cntp_ref, win_ref, cnt_ref):
    win_ref[...] = jnp.max(winp_ref[...], axis=0, keepdims=True)
    cnt_ref[...] = jnp.sum(cntp_ref[...], axis=0, keepdims=True)


def _combine(win_parts, cnt_parts):
    N = win_parts.shape[1]
    full = lambda shape: pl.BlockSpec(shape, lambda: (0, 0))
    return pl.pallas_call(
        _combine_body,
        in_specs=[full((_NW, N)), full((_NC, N))],
        out_specs=[full((1, N)), full((1, N))],
        out_shape=[jax.ShapeDtypeStruct((1, N), jnp.int32),
                   jax.ShapeDtypeStruct((1, N), jnp.float32)],
    )(win_parts, cnt_parts)


def _node_mm_body(h_ref, p_ref, Wlin_ref, blin_ref, Ws_ref, bs_ref, Wr_ref,
                  br_ref, Wp1_ref, bp1_ref, Wp2_ref, bp2_ref,
                  linh_ref, hpWs_ref, hpWr_ref, pWp1_ref, pWp2_ref):
    hb = h_ref[...]
    pb = p_ref[...]
    hp = jnp.concatenate([hb, pb], axis=1)
    f32 = jnp.float32
    linh_ref[...] = jnp.dot(hb, Wlin_ref[...], preferred_element_type=f32) + blin_ref[...]
    hpWs_ref[...] = jnp.dot(hp, Ws_ref[...], preferred_element_type=f32) + bs_ref[...]
    hpWr_ref[...] = jnp.dot(hp, Wr_ref[...], preferred_element_type=f32) + br_ref[...]
    pWp1_ref[...] = jnp.dot(pb, Wp1_ref[...], preferred_element_type=f32) + bp1_ref[...]
    pWp2_ref[...] = jnp.dot(pb, Wp2_ref[...], preferred_element_type=f32) + bp2_ref[...]


def _node_matmuls(h, p, W_lin, b_lin, Ws, bs, Wr, br, Wp1, bp1, Wp2, bp2):
    N, D = h.shape
    R = 1000
    grid = (N // R,)
    row = pl.BlockSpec((R, D), lambda i: (i, 0))
    row2 = pl.BlockSpec((R, 2 * D), lambda i: (i, 0))
    wfull = lambda shape: pl.BlockSpec(shape, lambda i: (0, 0))
    out_sd = jax.ShapeDtypeStruct((N, D), jnp.float32)
    return pl.pallas_call(
        _node_mm_body,
        grid=grid,
        in_specs=[row, row,
                  wfull((D, D)), wfull((1, D)),
                  wfull((2 * D, D)), wfull((1, D)),
                  wfull((2 * D, D)), wfull((1, D)),
                  wfull((D, D)), wfull((1, D)),
                  wfull((D, D)), wfull((1, D))],
        out_specs=[row, row, row, row, row],
        out_shape=[out_sd] * 5,
    )(h, p, W_lin, b_lin.reshape(1, D), Ws, bs.reshape(1, D),
      Wr, br.reshape(1, D), Wp1, bp1.reshape(1, D), Wp2, bp2.reshape(1, D))


def _edge_eta_body(g_ref, e_ref, Wlin_ref, b3_ref,
                   etanew_ref, s_ref, sum_ref, sumsq_ref):
    x = g_ref[...] + jnp.dot(e_ref[...], Wlin_ref[...],
                             preferred_element_type=jnp.float32) + b3_ref[...]
    eta = jax.nn.sigmoid(x)
    s = jnp.sum(eta, axis=1, keepdims=True)
    etanew_ref[...] = eta / s
    s_ref[...] = s
    bsum = jnp.sum(eta, axis=0, keepdims=True)
    bsq = jnp.sum(eta * eta, axis=0, keepdims=True)

    @pl.when(pl.program_id(0) == 0)
    def _init():
        sum_ref[...] = bsum
        sumsq_ref[...] = bsq

    @pl.when(pl.program_id(0) != 0)
    def _acc():
        sum_ref[...] += bsum
        sumsq_ref[...] += bsq


def _edge_eta(g, e, W_lin, b_lin):
    E, D = e.shape
    BE = 1000
    grid = (E // BE,)
    row = pl.BlockSpec((BE, D), lambda i: (i, 0))
    col = pl.BlockSpec((BE, 1), lambda i: (i, 0))
    wfull = lambda shape: pl.BlockSpec(shape, lambda i: (0, 0))
    return pl.pallas_call(
        _edge_eta_body,
        grid=grid,
        in_specs=[row, row, wfull((D, D)), wfull((1, D))],
        out_specs=[row, col, wfull((1, D)), wfull((1, D))],
        out_shape=[jax.ShapeDtypeStruct((E, D), jnp.float32),
                   jax.ShapeDtypeStruct((E, 1), jnp.float32),
                   jax.ShapeDtypeStruct((1, D), jnp.float32),
                   jax.ShapeDtypeStruct((1, D), jnp.float32)],
    )(g, e, W_lin, b_lin.reshape(1, D))


def _stats_body(W0_ref, W1_ref, cnt_ref, win_ref, g2_ref, hpWs_ref,
                hpWr_ref, pWp2_ref, sume_ref, sumsqe_ref, n_edges_ref,
                seg1_ref, seg2_ref, mean1_ref, inv1_ref,
                meane_ref, inve_ref, wc_ref):
    E = n_edges_ref[0]
    Nn = n_edges_ref[1]
    Ef = E.astype(jnp.float32)
    W = W0_ref[...] + W1_ref[...]
    wc_ref[...] = jnp.clip(win_ref[...], 0, Nn - 1)
    seg1 = hpWr_ref[...] * W
    seg2 = pWp2_ref[...] * W
    seg1_ref[...] = seg1
    seg2_ref[...] = seg2
    cnt = cnt_ref[...]
    hpWs = hpWs_ref[...]
    g2 = g2_ref[...]
    A1 = jnp.sum(cnt * hpWs, axis=0, keepdims=True)
    B1 = jnp.sum(cnt * hpWs * hpWs, axis=0, keepdims=True)
    S1 = jnp.sum(seg1, axis=0, keepdims=True)
    C1 = jnp.sum(2.0 * g2 * seg1 + seg1 * seg1, axis=0, keepdims=True)
    part1 = A1 + S1
    part2 = B1 + C1

    @pl.when(pl.program_id(0) == 0)
    def _init():
        mean1_ref[...] = part1
        inv1_ref[...] = part2

    @pl.when(pl.program_id(0) != 0)
    def _acc():
        mean1_ref[...] += part1
        inv1_ref[...] += part2

    @pl.when(pl.program_id(0) == pl.num_programs(0) - 1)
    def _fin():
        mean1 = mean1_ref[...] / Ef
        var1 = inv1_ref[...] / Ef - mean1 * mean1
        mean1_ref[...] = mean1
        inv1_ref[...] = lax.rsqrt(var1 + _EPS)
        meane = sume_ref[...] / Ef
        vare = sumsqe_ref[...] / Ef - meane * meane
        meane_ref[...] = meane
        inve_ref[...] = lax.rsqrt(vare + _EPS)


def _stats(W0, W1, cnt, win, g2, hpWs, hpWr, pWp2, sum_eta, sumsq_eta, E):
    N, D = W0.shape
    R = 2000
    grid = (N // R,)
    row = pl.BlockSpec((R, D), lambda i: (i, 0))
    col = pl.BlockSpec((R, 1), lambda i: (i, 0))
    bc = pl.BlockSpec((1, D), lambda i: (0, 0))
    sd = jax.ShapeDtypeStruct
    return pl.pallas_call(
        _stats_body,
        grid=grid,
        in_specs=[row, row, col, col, row, row, row, row, bc, bc,
                  pl.BlockSpec(memory_space=pltpu.SMEM)],
        out_specs=[row, row, bc, bc, bc, bc, col],
        out_shape=[sd((N, D), jnp.float32), sd((N, D), jnp.float32),
                   sd((1, D), jnp.float32), sd((1, D), jnp.float32),
                   sd((1, D), jnp.float32), sd((1, D), jnp.float32),
                   sd((N, 1), jnp.int32)],
    )(W0, W1, cnt.reshape(N, 1), win.reshape(N, 1), g2, hpWs, hpWr, pWp2,
      sum_eta, sumsq_eta, jnp.array([E, N], dtype=jnp.int32))


def _node_out_body(h_ref, p_ref, hpWs_ref, pWp1_ref, s1g_ref, s2g_ref,
                   win_ref, mean1_ref, inv1_ref, gamma_ref, beta_ref,
                   n_ref, hout_ref, pout_ref):
    Nn = n_ref[0]
    win = win_ref[...]
    has = win >= 0
    use = jnp.logical_and(has, win < Nn)
    h = h_ref[...]
    p = p_ref[...]
    x1 = hpWs_ref[...] + jnp.where(use, s1g_ref[...], 0.0)
    bn1 = (x1 - mean1_ref[...]) * inv1_ref[...] * gamma_ref[...] + beta_ref[...]
    hn = h + jnp.maximum(bn1, 0.0)
    hout_ref[...] = jnp.where(has, hn, h)
    x2 = pWp1_ref[...] + jnp.where(use, s2g_ref[...], 0.0)
    pout_ref[...] = jnp.where(has, p + jnp.tanh(x2), p)


def _node_out(h, p, hpWs, pWp1, s1g, s2g, win, mean1, inv1, gamma, beta, N_dim):
    N, D = h.shape
    R = 1000
    grid = (N // R,)
    row = pl.BlockSpec((R, D), lambda i: (i, 0))
    col = pl.BlockSpec((R, 1), lambda i: (i, 0))
    bc = lambda shape: pl.BlockSpec(shape, lambda i: (0, 0))
    sd = jax.ShapeDtypeStruct
    return pl.pallas_call(
        _node_out_body,
        grid=grid,
        in_specs=[row, row, row, row, row, row, col,
                  bc((1, D)), bc((1, D)), bc((1, D)), bc((1, D)),
                  pl.BlockSpec(memory_space=pltpu.SMEM)],
        out_specs=[row, row],
        out_shape=[sd((N, D), jnp.float32), sd((N, D), jnp.float32)],
    )(h, p, hpWs, pWp1, s1g, s2g, win.reshape(N, 1), mean1, inv1,
      gamma.reshape(1, D), beta.reshape(1, D),
      jnp.array([N_dim], dtype=jnp.int32))


def _e_out_body(e_ref, en_ref, s_ref, meane_ref, inve_ref, gamma_ref,
                beta_ref, eout_ref):
    eta = en_ref[...] * s_ref[...]
    bn = (eta - meane_ref[...]) * inve_ref[...] * gamma_ref[...] + beta_ref[...]
    eout_ref[...] = e_ref[...] + jnp.maximum(bn, 0.0)


def _e_out(e, eta_new, s, meane, inve, gamma, beta):
    E, D = e.shape
    BE = 1000
    grid = (E // BE,)
    row = pl.BlockSpec((BE, D), lambda i: (i, 0))
    col = pl.BlockSpec((BE, 1), lambda i: (i, 0))
    bc = lambda shape: pl.BlockSpec(shape, lambda i: (0, 0))
    return pl.pallas_call(
        _e_out_body,
        grid=grid,
        in_specs=[row, row, col, bc((1, D)), bc((1, D)), bc((1, D)), bc((1, D))],
        out_specs=row,
        out_shape=jax.ShapeDtypeStruct((E, D), jnp.float32),
    )(e, eta_new, s, meane, inve, gamma.reshape(1, D), beta.reshape(1, D))


def kernel(h, e, p, edge_index, W_lin, b_lin, Ws, bs, Wr, br, Wp1, bp1,
           Wp2, bp2, gamma, beta):
    N, D = h.shape
    E = e.shape[0]
    send = edge_index[0]
    rec = edge_index[1]

    lin_h, hpWs, hpWr, pWp1, pWp2 = _node_matmuls(
        h, p, W_lin, b_lin, Ws, bs, Wr, br, Wp1, bp1, Wp2, bp2)

    zeros_n = jnp.zeros((N,), jnp.float32)
    g, g2, win_parts, cnt_parts = _sc_gather_make(N, D, E)(
        lin_h, hpWs, send, rec, zeros_n)
    win2, cnt2 = _combine(win_parts, cnt_parts)
    win = win2.reshape(N)
    cnt = cnt2.reshape(N)

    eta_new, s, sum_eta, sumsq_eta = _edge_eta(g, e, W_lin, b_lin)

    W_parts = _sc_segsum_make(N, D, E)(eta_new, rec)

    seg1, seg2, mean1, inv1, meane, inve, wc = _stats(
        W_parts[0], W_parts[1], cnt, win, g2, hpWs, hpWr, pWp2,
        sum_eta, sumsq_eta, E)

    s1g, s2g = _sc_gather2_make(N, D)(seg1, seg2, wc.reshape(N))

    h_out, p_out = _node_out(h, p, hpWs, pWp1, s1g, s2g, win, mean1, inv1,
                             gamma, beta, N)
    e_out = _e_out(e, eta_new, s, meane, inve, gamma, beta)
    return (h_out, e_out, p_out)


# speedup vs baseline: 1.9464x; 1.9284x over previous
import functools

import jax
import jax.numpy as jnp
from jax import lax
from jax.experimental import pallas as pl
from jax.experimental.pallas import tpu as pltpu
from jax.experimental.pallas import tpu_sc as plsc

_EPS = 1e-5
_NC, _NS = 2, 16
_NW = _NC * _NS
_K = 128


def _sc_gather_make(N, D, E):
    NCH = E // _K
    NCH2 = N // _K
    REM2 = N - NCH2 * _K
    mesh = plsc.VectorSubcoreMesh(core_axis_name="c", subcore_axis_name="s")
    sd = jax.ShapeDtypeStruct

    @functools.partial(
        pl.kernel,
        out_type=[sd((E, D), jnp.float32),
                  sd((N, D), jnp.float32),
                  sd((_NW, N), jnp.int32),
                  sd((_NC, N), jnp.float32)],
        mesh=mesh,
    scratch_types=[
            pltpu.VMEM((_K,), jnp.int32),
            pltpu.VMEM((_K,), jnp.int32),
            pltpu.VMEM((_K,), jnp.int32),
            pltpu.VMEM((_K,), jnp.int32),
            pltpu.VMEM((_K, D), jnp.float32),
            pltpu.VMEM((_K, D), jnp.float32),
            pltpu.VMEM((_K, D), jnp.float32),
            pltpu.VMEM((_K, D), jnp.float32),
            pltpu.VMEM((_K,), jnp.int32),
            pltpu.VMEM((_K,), jnp.int32),
            pltpu.VMEM((_K,), jnp.int32),
            pltpu.VMEM((_K,), jnp.int32),
            pltpu.VMEM((_K,), jnp.int32),
            pltpu.VMEM((_K,), jnp.float32),
            pltpu.VMEM((N,), jnp.int32),
            pltpu.VMEM_SHARED((_NS * N,), jnp.int32),
            pltpu.VMEM_SHARED((N,), jnp.float32),
            pltpu.SemaphoreType.DMA,
            pltpu.SemaphoreType.DMA,
            pltpu.SemaphoreType.DMA,
            pltpu.SemaphoreType.DMA,
            pltpu.SemaphoreType.DMA,
            pltpu.SemaphoreType.DMA,
        ],
    )
    def k(linh_hbm, hpws_hbm, send_hbm, rec_hbm, zeros_hbm,
          g_hbm, g2_hbm, win_hbm, cnt_hbm,
          sidx0, sidx1, ridx0, ridx1, rowsa0, rowsa1, rowsb0, rowsb1,
          eid0, eid1, sidx20, sidx21, neg_v, ones_v,
          win_v, win_sh, cnt_sh, sema0, sema1, semb0, semb1, semo0, semo1):
        sidx_s = (sidx0, sidx1)
        ridx_s = (ridx0, ridx1)
        rowsa_s = (rowsa0, rowsa1)
        rowsb_s = (rowsb0, rowsb1)
        eid_s = (eid0, eid1)
        sidx2_s = (sidx20, sidx21)
        sema_s = (sema0, sema1)
        semb_s = (semb0, semb1)
        semo_s = (semo0, semo1)
        cid = lax.axis_index("c")
        sid = lax.axis_index("s")
        wid = sid * _NC + cid

        @pl.when(sid == 0)
        def _init_cnt():
            pltpu.sync_copy(zeros_hbm, cnt_sh)

        for i in range(_K // 16):
            sl = pl.ds(i * 16, 16)
            ones_v[sl] = jnp.ones((16,), jnp.float32)
            neg_v[sl] = jnp.full((16,), -1, jnp.int32)

        def _fill_win(j, c):
            pltpu.sync_copy(neg_v, win_sh.at[pl.ds(sid * N + j * _K, _K)])
            return c
        lax.fori_loop(0, N // _K, _fill_win, 0)
        if N % _K > 0:
            pltpu.sync_copy(neg_v.at[pl.ds(0, N % _K)],
                            win_sh.at[pl.ds(sid * N + (N // _K) * _K, N % _K)])
        plsc.subcore_barrier()

        n_i = (NCH - wid + _NW - 1) // _NW
        cbase = lambda t: (wid + t * _NW) * _K

        @pl.when(n_i > 0)
        def _prologue():
            pltpu.sync_copy(send_hbm.at[pl.ds(cbase(0), _K)], sidx0)
            pltpu.sync_copy(rec_hbm.at[pl.ds(cbase(0), _K)], ridx0)
            pltpu.async_copy(linh_hbm.at[sidx0], rowsa0, sema0)
            pltpu.async_copy(linh_hbm.at[ridx0], rowsb0, semb0)

        def _pair(j, c):
            for b in (0, 1):
                t = 2 * j + b
                bn = 1 - b

                @pl.when(t < n_i)
                def _step():
                    base = cbase(t)

                    @pl.when(t + 1 < n_i)
                    def _prefetch():
                        nbase = cbase(t + 1)
                        pltpu.sync_copy(send_hbm.at[pl.ds(nbase, _K)],
                                        sidx_s[bn])
                        pltpu.sync_copy(rec_hbm.at[pl.ds(nbase, _K)],
                                        ridx_s[bn])

                        @pl.when(t + 1 >= 2)
                        def _drain_prev():
                            pltpu.make_async_copy(
                                rowsa_s[bn], g_hbm.at[pl.ds(base, _K)],
                                semo_s[bn]).wait()
                        pltpu.async_copy(linh_hbm.at[sidx_s[bn]],
                                         rowsa_s[bn], sema_s[bn])
                        pltpu.async_copy(linh_hbm.at[ridx_s[bn]],
                                         rowsb_s[bn], semb_s[bn])

                    pltpu.make_async_copy(linh_hbm.at[sidx_s[b]],
                                          rowsa_s[b], sema_s[b]).wait()
                    pltpu.make_async_copy(linh_hbm.at[ridx_s[b]],
                                          rowsb_s[b], semb_s[b]).wait()
                    ra = rowsa_s[b]
                    rb = rowsb_s[b]

                    def _radd(r, cc):
                        for rr in range(2):
                            for c8 in range(D // 16):
                                sl = pl.ds(c8 * 16, 16)
                                ra[2 * r + rr, sl] = (ra[2 * r + rr, sl]
                                                      + rb[2 * r + rr, sl])
                        return cc
                    lax.fori_loop(0, _K // 2, _radd, 0)
                    pltpu.async_copy(ra, g_hbm.at[pl.ds(base, _K)],
                                     semo_s[b])
                    for r8 in range(_K // 16):
                        sl = pl.ds(r8 * 16, 16)
                        eid_s[b][sl] = base + r8 * 16 + lax.iota(jnp.int32, 16)
                        sidx2_s[b][sl] = sidx_s[b][sl] + sid * N
                    pltpu.sync_copy(eid_s[b], win_sh.at[sidx2_s[b]])
                    pltpu.sync_copy(ones_v, cnt_sh.at[sidx_s[b]], add=True)
            return c

        lax.fori_loop(0, (n_i + 1) // 2, _pair, 0)
        for b in (0, 1):
            lb = jnp.where((n_i - 1) % 2 == b, n_i - 1, n_i - 2)

            @pl.when(lb >= 0)
            def _drain_tail():
                pltpu.make_async_copy(rowsa_s[b],
                                      g_hbm.at[pl.ds(cbase(0), _K)],
                                      semo_s[b]).wait()

        def _chunk2(i, c):
            base = (wid + i * _NW) * _K
            pltpu.sync_copy(send_hbm.at[pl.ds(base, _K)], sidx0)
            pltpu.async_copy(hpws_hbm.at[sidx0], rowsa0, sema0).wait()
            pltpu.sync_copy(rowsa0, g2_hbm.at[pl.ds(base, _K)])
            return c

        n_i2 = (NCH2 - wid + _NW - 1) // _NW
        lax.fori_loop(0, n_i2, _chunk2, 0)

        if REM2 > 0:
            @pl.when(wid == _NW - 1)
            def _tail():
                tb = NCH2 * _K
                pltpu.sync_copy(send_hbm.at[pl.ds(tb, REM2)],
                                sidx0.at[pl.ds(0, REM2)])
                pltpu.async_copy(hpws_hbm.at[sidx0.at[pl.ds(0, REM2)]],
                                 rowsa0.at[pl.ds(0, REM2)], sema0).wait()
                pltpu.sync_copy(rowsa0.at[pl.ds(0, REM2)],
                                g2_hbm.at[pl.ds(tb, REM2)])

        pltpu.sync_copy(win_sh.at[pl.ds(sid * N, N)], win_v)
        pltpu.sync_copy(win_v, win_hbm.at[wid])
        plsc.subcore_barrier()

        @pl.when(sid == 0)
        def _write_cnt():
            pltpu.sync_copy(cnt_sh, cnt_hbm.at[cid])

    return k


def _sc_segsum_make(N, D, E):
    NCH = E // _K
    ZR = 125
    mesh = plsc.VectorSubcoreMesh(core_axis_name="c", subcore_axis_name="s")

    @functools.partial(
        pl.kernel,
        out_type=jax.ShapeDtypeStruct((_NC, N, D), jnp.float32),
        mesh=mesh,
        scratch_types=[
            pltpu.VMEM((_K,), jnp.int32),
            pltpu.VMEM((_K,), jnp.int32),
            pltpu.VMEM((_K, D), jnp.float32),
            pltpu.VMEM((_K, D), jnp.float32),
            pltpu.VMEM((ZR, D), jnp.float32),
            pltpu.VMEM_SHARED((N, D), jnp.float32),
            pltpu.SemaphoreType.DMA,
            pltpu.SemaphoreType.DMA,
        ],
    )
    def k(etanew_hbm, rec_hbm, w_hbm, ridx0, ridx1, env0, env1, zv, w_sh,
          seme0, seme1):
        ridx_s = (ridx0, ridx1)
        env_s = (env0, env1)
        seme_s = (seme0, seme1)
        cid = lax.axis_index("c")
        sid = lax.axis_index("s")
        wid = sid * _NC + cid

        def _zrow(r, c):
            for c8 in range(D // 16):
                zv[r, pl.ds(c8 * 16, 16)] = jnp.zeros((16,), jnp.float32)
            return c
        lax.fori_loop(0, ZR, _zrow, 0)
        nstripe = N // (_NS * ZR)
        def _zcp(j, c):
            pltpu.sync_copy(zv, w_sh.at[pl.ds((sid * nstripe + j) * ZR, ZR)])
            return c
        lax.fori_loop(0, nstripe, _zcp, 0)
        plsc.subcore_barrier()

        n_i = (NCH - wid + _NW - 1) // _NW
        cbase = lambda t: (wid + t * _NW) * _K

        @pl.when(n_i > 0)
        def _prologue():
            pltpu.sync_copy(rec_hbm.at[pl.ds(cbase(0), _K)], ridx0)
            pltpu.async_copy(etanew_hbm.at[pl.ds(cbase(0), _K)], env0, seme0)

        def _pair(j, c):
            for b in (0, 1):
                t = 2 * j + b
                bn = 1 - b

                @pl.when(t < n_i)
                def _step():
                    @pl.when(t + 1 < n_i)
                    def _prefetch():
                        nbase = cbase(t + 1)
                        pltpu.sync_copy(rec_hbm.at[pl.ds(nbase, _K)],
                                        ridx_s[bn])
                        pltpu.async_copy(etanew_hbm.at[pl.ds(nbase, _K)],
                                         env_s[bn], seme_s[bn])

                    pltpu.make_async_copy(etanew_hbm.at[pl.ds(cbase(t), _K)],
                                          env_s[b], seme_s[b]).wait()
                    pltpu.sync_copy(env_s[b], w_sh.at[ridx_s[b]], add=True)
            return c

        lax.fori_loop(0, (n_i + 1) // 2, _pair, 0)
        plsc.subcore_barrier()

        @pl.when(sid == 0)
        def _write():
            pltpu.sync_copy(w_sh, w_hbm.at[cid])

    return k


def _sc_gather2_make(N, D):
    NCH2 = N // _K
    REM2 = N - NCH2 * _K
    mesh = plsc.VectorSubcoreMesh(core_axis_name="c", subcore_axis_name="s")
    sd = jax.ShapeDtypeStruct

    @functools.partial(
        pl.kernel,
        out_type=[sd((N, D), jnp.float32), sd((N, D), jnp.float32)],
        mesh=mesh,
        scratch_types=[
            pltpu.VMEM((_K,), jnp.int32),
            pltpu.VMEM((_K,), jnp.int32),
            pltpu.VMEM((_K, D), jnp.float32),
            pltpu.VMEM((_K, D), jnp.float32),
            pltpu.VMEM((_K, D), jnp.float32),
            pltpu.VMEM((_K, D), jnp.float32),
            pltpu.SemaphoreType.DMA,
            pltpu.SemaphoreType.DMA,
            pltpu.SemaphoreType.DMA,
            pltpu.SemaphoreType.DMA,
        ],
    )
    def k(seg1_hbm, seg2_hbm, wc_hbm, s1g_hbm, s2g_hbm,
          idx0, idx1, r10, r11, r20, r21, semg0, semg1, semo0, semo1):
        idx_s = (idx0, idx1)
        r1_s = (r10, r11)
        r2_s = (r20, r21)
        semg_s = (semg0, semg1)
        semo_s = (semo0, semo1)
        cid = lax.axis_index("c")
        sid = lax.axis_index("s")
        wid = sid * _NC + cid

        n_i = (NCH2 - wid + _NW - 1) // _NW
        cbase = lambda t: (wid + t * _NW) * _K

        @pl.when(n_i > 0)
        def _prologue():
            pltpu.sync_copy(wc_hbm.at[pl.ds(cbase(0), _K)], idx0)
            pltpu.async_copy(seg1_hbm.at[idx0], r10, semg0)
            pltpu.async_copy(seg2_hbm.at[idx0], r20, semg0)

        def _pair(j, c):
            for b in (0, 1):
                t = 2 * j + b
                bn = 1 - b

                @pl.when(t < n_i)
                def _step():
                    base = cbase(t)

                    @pl.when(t + 1 < n_i)
                    def _prefetch():
                        nbase = cbase(t + 1)

                        @pl.when(t + 1 >= 2)
                        def _drain_prev():
                            pltpu.make_async_copy(
                                r1_s[bn], s1g_hbm.at[pl.ds(base, _K)],
                                semo_s[bn]).wait()
                            pltpu.make_async_copy(
                                r2_s[bn], s2g_hbm.at[pl.ds(base, _K)],
                                semo_s[bn]).wait()
                        pltpu.sync_copy(wc_hbm.at[pl.ds(nbase, _K)],
                                        idx_s[bn])
                        pltpu.async_copy(seg1_hbm.at[idx_s[bn]],
                                         r1_s[bn], semg_s[bn])
                        pltpu.async_copy(seg2_hbm.at[idx_s[bn]],
                                         r2_s[bn], semg_s[bn])

                    pltpu.make_async_copy(seg1_hbm.at[idx_s[b]],
                                          r1_s[b], semg_s[b]).wait()
                    pltpu.make_async_copy(seg2_hbm.at[idx_s[b]],
                                          r2_s[b], semg_s[b]).wait()
                    pltpu.async_copy(r1_s[b], s1g_hbm.at[pl.ds(base, _K)],
                                     semo_s[b])
                    pltpu.async_copy(r2_s[b], s2g_hbm.at[pl.ds(base, _K)],
                                     semo_s[b])
            return c

        lax.fori_loop(0, (n_i + 1) // 2, _pair, 0)
        for b in (0, 1):
            lb = jnp.where((n_i - 1) % 2 == b, n_i - 1, n_i - 2)

            @pl.when(lb >= 0)
            def _drain_tail():
                pltpu.make_async_copy(r1_s[b], s1g_hbm.at[pl.ds(cbase(0), _K)],
                                      semo_s[b]).wait()
                pltpu.make_async_copy(r2_s[b], s2g_hbm.at[pl.ds(cbase(0), _K)],
                                      semo_s[b]).wait()

        if REM2 > 0:
            @pl.when(wid == _NW - 1)
            def _tail():
                tb = NCH2 * _K
                pltpu.sync_copy(wc_hbm.at[pl.ds(tb, REM2)],
                                idx0.at[pl.ds(0, REM2)])
                cp_a = pltpu.async_copy(seg1_hbm.at[idx0.at[pl.ds(0, REM2)]],
                                        r10.at[pl.ds(0, REM2)], semg0)
                cp_b = pltpu.async_copy(seg2_hbm.at[idx0.at[pl.ds(0, REM2)]],
                                        r20.at[pl.ds(0, REM2)], semg0)
                cp_a.wait()
                cp_b.wait()
                pltpu.sync_copy(r10.at[pl.ds(0, REM2)],
                                s1g_hbm.at[pl.ds(tb, REM2)])
                pltpu.sync_copy(r20.at[pl.ds(0, REM2)],
                                s2g_hbm.at[pl.ds(tb, REM2)])

    return k


def _combine_body(winp_ref, cntp_ref, win_ref, cnt_ref):
    win_ref[...] = jnp.max(winp_ref[...], axis=0, keepdims=True)
    cnt_ref[...] = jnp.sum(cntp_ref[...], axis=0, keepdims=True)


def _combine(win_parts, cnt_parts):
    N = win_parts.shape[1]
    full = lambda shape: pl.BlockSpec(shape, lambda: (0, 0))
    return pl.pallas_call(
        _combine_body,
        in_specs=[full((_NW, N)), full((_NC, N))],
        out_specs=[full((1, N)), full((1, N))],
        out_shape=[jax.ShapeDtypeStruct((1, N), jnp.int32),
                   jax.ShapeDtypeStruct((1, N), jnp.float32)],
    )(win_parts, cnt_parts)


def _node_mm_body(h_ref, p_ref, Wlin_ref, blin_ref, Ws_ref, bs_ref, Wr_ref,
                  br_ref, Wp1_ref, bp1_ref, Wp2_ref, bp2_ref,
                  linh_ref, hpWs_ref, hpWr_ref, pWp1_ref, pWp2_ref):
    hb = h_ref[...]
    pb = p_ref[...]
    hp = jnp.concatenate([hb, pb], axis=1)
    f32 = jnp.float32
    linh_ref[...] = jnp.dot(hb, Wlin_ref[...], preferred_element_type=f32) + blin_ref[...]
    hpWs_ref[...] = jnp.dot(hp, Ws_ref[...], preferred_element_type=f32) + bs_ref[...]
    hpWr_ref[...] = jnp.dot(hp, Wr_ref[...], preferred_element_type=f32) + br_ref[...]
    pWp1_ref[...] = jnp.dot(pb, Wp1_ref[...], preferred_element_type=f32) + bp1_ref[...]
    pWp2_ref[...] = jnp.dot(pb, Wp2_ref[...], preferred_element_type=f32) + bp2_ref[...]


def _node_matmuls(h, p, W_lin, b_lin, Ws, bs, Wr, br, Wp1, bp1, Wp2, bp2):
    N, D = h.shape
    R = 1000
    grid = (N // R,)
    row = pl.BlockSpec((R, D), lambda i: (i, 0))
    row2 = pl.BlockSpec((R, 2 * D), lambda i: (i, 0))
    wfull = lambda shape: pl.BlockSpec(shape, lambda i: (0, 0))
    out_sd = jax.ShapeDtypeStruct((N, D), jnp.float32)
    return pl.pallas_call(
        _node_mm_body,
        grid=grid,
        in_specs=[row, row,
                  wfull((D, D)), wfull((1, D)),
                  wfull((2 * D, D)), wfull((1, D)),
                  wfull((2 * D, D)), wfull((1, D)),
                  wfull((D, D)), wfull((1, D)),
                  wfull((D, D)), wfull((1, D))],
        out_specs=[row, row, row, row, row],
        out_shape=[out_sd] * 5,
    )(h, p, W_lin, b_lin.reshape(1, D), Ws, bs.reshape(1, D),
      Wr, br.reshape(1, D), Wp1, bp1.reshape(1, D), Wp2, bp2.reshape(1, D))


def _edge_eta_body(g_ref, e_ref, Wlin_ref, b3_ref,
                   etanew_ref, s_ref, sum_ref, sumsq_ref):
    x = g_ref[...] + jnp.dot(e_ref[...], Wlin_ref[...],
                             preferred_element_type=jnp.float32) + b3_ref[...]
    eta = jax.nn.sigmoid(x)
    s = jnp.sum(eta, axis=1, keepdims=True)
    etanew_ref[...] = eta / s
    s_ref[...] = s
    bsum = jnp.sum(eta, axis=0, keepdims=True)
    bsq = jnp.sum(eta * eta, axis=0, keepdims=True)

    @pl.when(pl.program_id(0) == 0)
    def _init():
        sum_ref[...] = bsum
        sumsq_ref[...] = bsq

    @pl.when(pl.program_id(0) != 0)
    def _acc():
        sum_ref[...] += bsum
        sumsq_ref[...] += bsq


def _edge_eta(g, e, W_lin, b_lin):
    E, D = e.shape
    BE = 1000
    grid = (E // BE,)
    row = pl.BlockSpec((BE, D), lambda i: (i, 0))
    col = pl.BlockSpec((BE, 1), lambda i: (i, 0))
    wfull = lambda shape: pl.BlockSpec(shape, lambda i: (0, 0))
    return pl.pallas_call(
        _edge_eta_body,
        grid=grid,
        in_specs=[row, row, wfull((D, D)), wfull((1, D))],
        out_specs=[row, col, wfull((1, D)), wfull((1, D))],
        out_shape=[jax.ShapeDtypeStruct((E, D), jnp.float32),
                   jax.ShapeDtypeStruct((E, 1), jnp.float32),
                   jax.ShapeDtypeStruct((1, D), jnp.float32),
                   jax.ShapeDtypeStruct((1, D), jnp.float32)],
    )(g, e, W_lin, b_lin.reshape(1, D))


def _stats_body(W0_ref, W1_ref, cnt_ref, win_ref, g2_ref, hpWs_ref,
                hpWr_ref, pWp2_ref, sume_ref, sumsqe_ref, n_edges_ref,
                seg1_ref, seg2_ref, mean1_ref, inv1_ref,
                meane_ref, inve_ref, wc_ref):
    E = n_edges_ref[0]
    Nn = n_edges_ref[1]
    Ef = E.astype(jnp.float32)
    W = W0_ref[...] + W1_ref[...]
    win = win_ref[...]
    R = win.shape[0]
    rows = (pl.program_id(0) * R
            + lax.broadcasted_iota(jnp.int32, win.shape, 0))
    wc_ref[...] = jnp.where((win >= 0) & (win < Nn), win, rows)
    seg1 = hpWr_ref[...] * W
    seg2 = pWp2_ref[...] * W
    seg1_ref[...] = seg1
    seg2_ref[...] = seg2
    cnt = cnt_ref[...]
    hpWs = hpWs_ref[...]
    g2 = g2_ref[...]
    A1 = jnp.sum(cnt * hpWs, axis=0, keepdims=True)
    B1 = jnp.sum(cnt * hpWs * hpWs, axis=0, keepdims=True)
    S1 = jnp.sum(seg1, axis=0, keepdims=True)
    C1 = jnp.sum(2.0 * g2 * seg1 + seg1 * seg1, axis=0, keepdims=True)
    part1 = A1 + S1
    part2 = B1 + C1

    @pl.when(pl.program_id(0) == 0)
    def _init():
        mean1_ref[...] = part1
        inv1_ref[...] = part2

    @pl.when(pl.program_id(0) != 0)
    def _acc():
        mean1_ref[...] += part1
        inv1_ref[...] += part2

    @pl.when(pl.program_id(0) == pl.num_programs(0) - 1)
    def _fin():
        mean1 = mean1_ref[...] / Ef
        var1 = inv1_ref[...] / Ef - mean1 * mean1
        mean1_ref[...] = mean1
        inv1_ref[...] = lax.rsqrt(var1 + _EPS)
        meane = sume_ref[...] / Ef
        vare = sumsqe_ref[...] / Ef - meane * meane
        meane_ref[...] = meane
        inve_ref[...] = lax.rsqrt(vare + _EPS)


def _stats(W0, W1, cnt, win, g2, hpWs, hpWr, pWp2, sum_eta, sumsq_eta, E):
    N, D = W0.shape
    R = 2000
    grid = (N // R,)
    row = pl.BlockSpec((R, D), lambda i: (i, 0))
    col = pl.BlockSpec((R, 1), lambda i: (i, 0))
    bc = pl.BlockSpec((1, D), lambda i: (0, 0))
    sd = jax.ShapeDtypeStruct
    return pl.pallas_call(
        _stats_body,
        grid=grid,
        in_specs=[row, row, col, col, row, row, row, row, bc, bc,
                  pl.BlockSpec(memory_space=pltpu.SMEM)],
        out_specs=[row, row, bc, bc, bc, bc, col],
        out_shape=[sd((N, D), jnp.float32), sd((N, D), jnp.float32),
                   sd((1, D), jnp.float32), sd((1, D), jnp.float32),
                   sd((1, D), jnp.float32), sd((1, D), jnp.float32),
                   sd((N, 1), jnp.int32)],
    )(W0, W1, cnt.reshape(N, 1), win.reshape(N, 1), g2, hpWs, hpWr, pWp2,
      sum_eta, sumsq_eta, jnp.array([E, N], dtype=jnp.int32))


def _node_out_body(h_ref, p_ref, hpWs_ref, pWp1_ref, s1g_ref, s2g_ref,
                   win_ref, mean1_ref, inv1_ref, gamma_ref, beta_ref,
                   n_ref, hout_ref, pout_ref):
    Nn = n_ref[0]
    win = win_ref[...]
    has = win >= 0
    use = jnp.logical_and(has, win < Nn)
    h = h_ref[...]
    p = p_ref[...]
    x1 = hpWs_ref[...] + jnp.where(use, s1g_ref[...], 0.0)
    bn1 = (x1 - mean1_ref[...]) * inv1_ref[...] * gamma_ref[...] + beta_ref[...]
    hn = h + jnp.maximum(bn1, 0.0)
    hout_ref[...] = jnp.where(has, hn, h)
    x2 = pWp1_ref[...] + jnp.where(use, s2g_ref[...], 0.0)
    pout_ref[...] = jnp.where(has, p + jnp.tanh(x2), p)


def _node_out(h, p, hpWs, pWp1, s1g, s2g, win, mean1, inv1, gamma, beta, N_dim):
    N, D = h.shape
    R = 1000
    grid = (N // R,)
    row = pl.BlockSpec((R, D), lambda i: (i, 0))
    col = pl.BlockSpec((R, 1), lambda i: (i, 0))
    bc = lambda shape: pl.BlockSpec(shape, lambda i: (0, 0))
    sd = jax.ShapeDtypeStruct
    return pl.pallas_call(
        _node_out_body,
        grid=grid,
        in_specs=[row, row, row, row, row, row, col,
                  bc((1, D)), bc((1, D)), bc((1, D)), bc((1, D)),
                  pl.BlockSpec(memory_space=pltpu.SMEM)],
        out_specs=[row, row],
        out_shape=[sd((N, D), jnp.float32), sd((N, D), jnp.float32)],
    )(h, p, hpWs, pWp1, s1g, s2g, win.reshape(N, 1), mean1, inv1,
      gamma.reshape(1, D), beta.reshape(1, D),
      jnp.array([N_dim], dtype=jnp.int32))


def _e_out_body(e_ref, en_ref, s_ref, meane_ref, inve_ref, gamma_ref,
                beta_ref, eout_ref):
    eta = en_ref[...] * s_ref[...]
    bn = (eta - meane_ref[...]) * inve_ref[...] * gamma_ref[...] + beta_ref[...]
    eout_ref[...] = e_ref[...] + jnp.maximum(bn, 0.0)


def _e_out(e, eta_new, s, meane, inve, gamma, beta):
    E, D = e.shape
    BE = 1000
    grid = (E // BE,)
    row = pl.BlockSpec((BE, D), lambda i: (i, 0))
    col = pl.BlockSpec((BE, 1), lambda i: (i, 0))
    bc = lambda shape: pl.BlockSpec(shape, lambda i: (0, 0))
    return pl.pallas_call(
        _e_out_body,
        grid=grid,
        in_specs=[row, row, col, bc((1, D)), bc((1, D)), bc((1, D)), bc((1, D))],
        out_specs=row,
        out_shape=jax.ShapeDtypeStruct((E, D), jnp.float32),
    )(e, eta_new, s, meane, inve, gamma.reshape(1, D), beta.reshape(1, D))


def kernel(h, e, p, edge_index, W_lin, b_lin, Ws, bs, Wr, br, Wp1, bp1,
           Wp2, bp2, gamma, beta):
    N, D = h.shape
    E = e.shape[0]
    send = edge_index[0]
    rec = edge_index[1]

    lin_h, hpWs, hpWr, pWp1, pWp2 = _node_matmuls(
        h, p, W_lin, b_lin, Ws, bs, Wr, br, Wp1, bp1, Wp2, bp2)

    zeros_n = jnp.zeros((N,), jnp.float32)
    g, g2, win_parts, cnt_parts = _sc_gather_make(N, D, E)(
        lin_h, hpWs, send, rec, zeros_n)
    win2, cnt2 = _combine(win_parts, cnt_parts)
    win = win2.reshape(N)
    cnt = cnt2.reshape(N)

    eta_new, s, sum_eta, sumsq_eta = _edge_eta(g, e, W_lin, b_lin)

    W_parts = _sc_segsum_make(N, D, E)(eta_new, rec)

    seg1, seg2, mean1, inv1, meane, inve, wc = _stats(
        W_parts[0], W_parts[1], cnt, win, g2, hpWs, hpWr, pWp2,
        sum_eta, sumsq_eta, E)

    s1g, s2g = _sc_gather2_make(N, D)(seg1, seg2, wc.reshape(N))

    h_out, p_out = _node_out(h, p, hpWs, pWp1, s1g, s2g, win, mean1, inv1,
                             gamma, beta, N)
    e_out = _e_out(e, eta_new, s, meane, inve, gamma, beta)
    return (h_out, e_out, p_out)
